# baseline (speedup 1.0000x reference)
import functools

import jax
import jax.numpy as jnp
from jax import lax
from jax.experimental import pallas as pl
from jax.experimental.pallas import tpu as pltpu
from jax.experimental.pallas import tpu_sc as plsc

N = 10000
E = 160000
F = 128
FE = 16
FG = 16
H = 256
G = 64

NC = 2
NS = 16
NW = NC * NS
CH = 128
EP = 163840
NCH = EP // (NW * CH)
SCH = EP // (NS * CH)
NPAD = 10240
NROW = NPAD // NS
BE = 512
BN = 1024

_f32 = jnp.float32
_i32 = jnp.int32


def _mesh():
    return plsc.VectorSubcoreMesh(core_axis_name="c", subcore_axis_name="s",
                                  num_cores=NC, num_subcores=NS)


def _count_body(col2, cntp, coli, buf, obuf, acc):
    c = lax.axis_index("c")
    s = lax.axis_index("s")
    w = s * NC + c
    zero16 = jnp.zeros((16,), _f32)
    one16 = jnp.ones((16,), _f32)

    def zb(i, carry):
        for j in range(F // 16):
            buf[i, pl.ds(j * 16, 16)] = zero16
            obuf[i, pl.ds(j * 16, 16)] = one16
        return carry

    lax.fori_loop(0, CH, zb, 0)

    def zc(k, carry):
        pltpu.sync_copy(buf, acc.at[pl.ds(s * NROW + k * CH, CH)])
        return carry

    lax.fori_loop(0, NROW // CH, zc, 0)
    plsc.subcore_barrier()
    pltpu.sync_copy(col2.at[pl.ds(w * NCH, NCH)], coli)

    def step(i, carry):
        pltpu.sync_copy(obuf, acc.at[coli.at[i]], add=True)
        return carry

    lax.fori_loop(0, NCH, step, 0)
    plsc.subcore_barrier()
    pltpu.sync_copy(acc.at[pl.ds(s * NROW, NROW)],
                    cntp.at[c, pl.ds(s * NROW, NROW)])


def _count_call(col2):
    k = pl.kernel(
        _count_body,
        out_type=jax.ShapeDtypeStruct((NC, NPAD, F), _f32),
        mesh=_mesh(),
        scratch_types=[
            pltpu.VMEM((NCH, CH), _i32),
            pltpu.VMEM((CH, F), _f32),
            pltpu.VMEM((CH, F), _f32),
            pltpu.VMEM_SHARED((NPAD, F), _f32),
        ],
    )
    return k(col2)


def _gidx_body(row2, b128, gw, rowi, bufg, s1):
    c = lax.axis_index("c")
    s = lax.axis_index("s")
    w = s * NC + c
    pltpu.sync_copy(row2.at[pl.ds(w * NCH, NCH)], rowi)

    def step(i, carry):
        ebase = (w * NCH + i) * CH
        pltpu.async_copy(b128.at[rowi.at[i]], bufg, s1).wait()
        pltpu.sync_copy(bufg, gw.at[pl.ds(ebase, CH)])
        return carry

    lax.fori_loop(0, NCH, step, 0)


def _gidx_call(row2, b128):
    k = pl.kernel(
        _gidx_body,
        out_type=jax.ShapeDtypeStruct((EP, F), _i32),
        mesh=_mesh(),
        scratch_types=[
            pltpu.VMEM((NCH, CH), _i32),
            pltpu.VMEM((CH, F), _i32),
            pltpu.SemaphoreType.DMA,
        ],
    )
    return k(row2, b128)


def _compact_body(gw, gi_o):
    gi_o[...] = gw[...][:, :1]


def _compact_call(gw):
    return pl.pallas_call(
        _compact_body,
        grid=(EP // BE,),
        in_specs=[pl.BlockSpec((BE, F), lambda i: (i, 0))],
        out_specs=pl.BlockSpec((BE, 1), lambda i: (i, 0)),
        out_shape=jax.ShapeDtypeStruct((EP, 1), _i32),
        compiler_params=pltpu.CompilerParams(
            dimension_semantics=("arbitrary",)),
    )(gw)


GA = 80
GPAD = 1280


def _gather_body(xtab, row2, col2, xr_o, xc_o,
                 rowi, coli, bxr0, bxc0, bxr1, bxc1, sr0, sc0, sr1, sc1):
    c = lax.axis_index("c")
    s = lax.axis_index("s")

    @pl.when(c == 0)
    def _():
        cbase = s * GA
        pltpu.sync_copy(row2.at[pl.ds(cbase, GA)], rowi)
        pltpu.sync_copy(col2.at[pl.ds(cbase, GA)], coli)

        def start(l, br, bc, svr, svc):
            pltpu.async_copy(xtab.at[rowi.at[l]], br, svr)
            pltpu.async_copy(xtab.at[coli.at[l]], bc, svc)

        def finish(l, br, bc, svr, svc):
            pltpu.make_async_copy(xtab.at[rowi.at[l]], br, svr).wait()
            pltpu.make_async_copy(xtab.at[coli.at[l]], bc, svc).wait()
            base = (cbase + l) * CH
            pltpu.sync_copy(br, xr_o.at[pl.ds(base, CH)])
            pltpu.sync_copy(bc, xc_o.at[pl.ds(base, CH)])

        start(0, bxr0, bxc0, sr0, sc0)
        start(1, bxr1, bxc1, sr1, sc1)

        def step(k, carry):
            i0 = 2 * k
            finish(i0, bxr0, bxc0, sr0, sc0)
            start(i0 + 2, bxr0, bxc0, sr0, sc0)
            finish(i0 + 1, bxr1, bxc1, sr1, sc1)
            start(i0 + 3, bxr1, bxc1, sr1, sc1)
            return carry

        lax.fori_loop(0, GA // 2 - 1, step, 0)
        finish(GA - 2, bxr0, bxc0, sr0, sc0)
        finish(GA - 1, bxr1, bxc1, sr1, sc1)


def _gather_call(xtab, row2p, col2p):
    k = pl.kernel(
        _gather_body,
        out_type=[
            jax.ShapeDtypeStruct((EP, F), _f32),
            jax.ShapeDtypeStruct((EP, F), _f32),
        ],
        mesh=_mesh(),
        scratch_types=[
            pltpu.VMEM((GA, CH), _i32),
            pltpu.VMEM((GA, CH), _i32),
            pltpu.VMEM((CH, F), _f32),
            pltpu.VMEM((CH, F), _f32),
            pltpu.VMEM((CH, F), _f32),
            pltpu.VMEM((CH, F), _f32),
            pltpu.SemaphoreType.DMA,
            pltpu.SemaphoreType.DMA,
            pltpu.SemaphoreType.DMA,
            pltpu.SemaphoreType.DMA,
        ],
    )
    return k(xtab, row2p, col2p)


def _scatter_body(h3, col2, s_out, coli, buf, acc, s1):
    c = lax.axis_index("c")
    s = lax.axis_index("s")
    zero16 = jnp.zeros((16,), _f32)

    def zb(i, carry):
        for j in range(F // 16):
            buf[i, pl.ds(j * 16, 16)] = zero16
        return carry

    lax.fori_loop(0, CH, zb, 0)

    def zc(k, carry):
        pltpu.sync_copy(buf, acc.at[pl.ds(s * NROW + k * CH, CH)])
        return carry

    lax.fori_loop(0, NROW // CH, zc, 0)
    plsc.subcore_barrier()

    pltpu.sync_copy(col2.at[pl.ds(s * SCH, SCH)], coli)

    def step(i, carry):
        gbase = (s * SCH + i) * CH
        pltpu.async_copy(h3.at[c, pl.ds(gbase, CH)], buf, s1).wait()
        pltpu.sync_copy(buf, acc.at[coli.at[i]], add=True)
        return carry

    lax.fori_loop(0, SCH, step, 0)
    plsc.subcore_barrier()
    pltpu.sync_copy(acc.at[pl.ds(s * NROW, NROW)],
                    s_out.at[c, pl.ds(s * NROW, NROW)])


def _scatter_call(h3, col2):
    k = pl.kernel(
        _scatter_body,
        out_type=jax.ShapeDtypeStruct((NC, NPAD, F), _f32),
        mesh=_mesh(),
        scratch_types=[
            pltpu.VMEM((SCH, CH), _i32),
            pltpu.VMEM((CH, F), _f32),
            pltpu.VMEM_SHARED((NPAD, F), _f32),
            pltpu.SemaphoreType.DMA,
        ],
    )
    return k(h3, col2)


def _prep_body(b2, cntp, u, wd, b1e, oh_o, p_o, cnt_o, gc_o):
    oh = (b2[...] == lax.broadcasted_iota(_i32, (1, G), 1)).astype(_f32)
    oh_o[...] = oh
    p_o[...] = jnp.dot(u[...], wd[...], preferred_element_type=_f32) + b1e[...]
    cnt_o[...] = cntp[0][:, :1] + cntp[1][:, :1]

    @pl.when(pl.program_id(0) == 0)
    def _():
        gc_o[...] = jnp.zeros_like(gc_o)

    gc_o[...] += lax.dot_general(oh, jnp.ones((BN, 1), _f32),
                                 (((0,), (0,)), ((), ())),
                                 preferred_element_type=_f32)


def _prep_call(b2, cntp, u, wd, b1e):
    grid = (NPAD // BN,)
    return pl.pallas_call(
        _prep_body,
        grid=grid,
        in_specs=[
            pl.BlockSpec((BN, 1), lambda i: (i, 0)),
            pl.BlockSpec((NC, BN, F), lambda i: (0, i, 0)),
            pl.BlockSpec((G, FG), lambda i: (0, 0)),
            pl.BlockSpec((FG, H), lambda i: (0, 0)),
            pl.BlockSpec((1, H), lambda i: (0, 0)),
        ],
        out_specs=[
            pl.BlockSpec((BN, G), lambda i: (i, 0)),
            pl.BlockSpec((G, H), lambda i: (0, 0)),
            pl.BlockSpec((BN, 1), lambda i: (i, 0)),
            pl.BlockSpec((G, 1), lambda i: (0, 0)),
        ],
        out_shape=[
            jax.ShapeDtypeStruct((NPAD, G), _f32),
            jax.ShapeDtypeStruct((G, H), _f32),
            jax.ShapeDtypeStruct((NPAD, 1), _f32),
            jax.ShapeDtypeStruct((G, 1), _f32),
        ],
        compiler_params=pltpu.CompilerParams(
            dimension_semantics=("arbitrary",)),
    )(b2, cntp, u, wd, b1e)


def _edge_body(xr, xc, gi, ea, wxx, wc, pmat, w2e, b2e, wna, wnb, b1n,
               ea_o, h3_o):
    xx = jnp.concatenate([xr[...], xc[...]], axis=1)
    ohe = (gi[...] == lax.broadcasted_iota(_i32, (1, G), 1)).astype(_f32)
    h = jnp.dot(xx, wxx[...], preferred_element_type=_f32)
    h += jnp.dot(ea[...], wc[...], preferred_element_type=_f32)
    h += jnp.dot(ohe, pmat[...], preferred_element_type=_f32)
    h = jnp.maximum(h, 0.0)
    ean = jnp.dot(h, w2e[...], preferred_element_type=_f32) + b2e[...]
    hn = jnp.dot(xr[...], wna[...], preferred_element_type=_f32)
    hn += jnp.dot(ean, wnb[...], preferred_element_type=_f32) + b1n[...]
    hn = jnp.maximum(hn, 0.0)
    m = pl.program_id(0) * BE + lax.broadcasted_iota(_i32, (BE, 1), 0) < E
    ea_o[...] = jnp.where(m, ean, 0.0)
    h3_o[0] = jnp.where(m, hn[:, :F], 0.0)
    h3_o[1] = jnp.where(m, hn[:, F:], 0.0)


def _edge_call(xr, xc, gi, ea, wxx, wc, pmat, w2e, b2e, wna, wnb, b1n):
    grid = (EP // BE,)
    wspec = lambda r, c: pl.BlockSpec((r, c), lambda i: (0, 0))
    return pl.pallas_call(
        _edge_body,
        grid=grid,
        in_specs=[
            pl.BlockSpec((BE, F), lambda i: (i, 0)),
            pl.BlockSpec((BE, F), lambda i: (i, 0)),
            pl.BlockSpec((BE, 1), lambda i: (i, 0)),
            pl.BlockSpec((BE, FE), lambda i: (i, 0)),
            wspec(2 * F, H), wspec(FE, H), wspec(G, H),
            wspec(H, FE), wspec(1, FE),
            wspec(F, H), wspec(FE, H), wspec(1, H),
        ],
        out_specs=[
            pl.BlockSpec((BE, FE), lambda i: (i, 0)),
            pl.BlockSpec((NC, BE, F), lambda i: (0, i, 0)),
        ],
        out_shape=[
            jax.ShapeDtypeStruct((EP, FE), _f32),
            jax.ShapeDtypeStruct((NC, EP, F), _f32),
        ],
        compiler_params=pltpu.CompilerParams(
            dimension_semantics=("arbitrary",)),
    )(xr, xc, gi, ea, wxx, wc, pmat, w2e, b2e, wna, wnb, b1n)


def _node_body(S, x, oh, cnt, u, w2a, w2b, b2n1, n2a, n2b, n2c, b1n2,
               n2w2, b2n2, xo, xmo):
    cnt_ = cnt[...]
    agg = jnp.dot(S[0], w2a[...], preferred_element_type=_f32)
    agg += jnp.dot(S[1], w2b[...], preferred_element_type=_f32)
    agg = (agg + cnt_ * b2n1[...]) / jnp.maximum(cnt_, 1.0)
    ub = jnp.dot(oh[...], u[...], preferred_element_type=_f32)
    t = jnp.dot(x[...], n2a[...], preferred_element_type=_f32)
    t += jnp.dot(agg, n2b[...], preferred_element_type=_f32)
    t += jnp.dot(ub, n2c[...], preferred_element_type=_f32) + b1n2[...]
    t = jnp.maximum(t, 0.0)
    xn = jnp.dot(t, n2w2[...], preferred_element_type=_f32) + b2n2[...]
    xo[...] = xn

    @pl.when(pl.program_id(0) == 0)
    def _():
        xmo[...] = jnp.zeros_like(xmo)

    xmo[...] += lax.dot_general(oh[...], xn, (((0,), (0,)), ((), ())),
                                preferred_element_type=_f32)


def _node_call(S, x, oh, cnt, u, w2a, w2b, b2n1, n2a, n2b, n2c, b1n2,
               n2w2, b2n2):
    grid = (NPAD // BN,)
    wspec = lambda r, c: pl.BlockSpec((r, c), lambda i: (0, 0))
    return pl.pallas_call(
        _node_body,
        grid=grid,
        in_specs=[
            pl.BlockSpec((NC, BN, F), lambda i: (0, i, 0)),
            pl.BlockSpec((BN, F), lambda i: (i, 0)),
            pl.BlockSpec((BN, G), lambda i: (i, 0)),
            pl.BlockSpec((BN, 1), lambda i: (i, 0)),
            wspec(G, FG),
            wspec(F, H), wspec(F, H), wspec(1, H),
            wspec(F, H), wspec(H, H), wspec(FG, H), wspec(1, H),
            wspec(H, F), wspec(1, F),
        ],
        out_specs=[
            pl.BlockSpec((BN, F), lambda i: (i, 0)),
            pl.BlockSpec((G, F), lambda i: (0, 0)),
        ],
        out_shape=[
            jax.ShapeDtypeStruct((NPAD, F), _f32),
            jax.ShapeDtypeStruct((G, F), _f32),
        ],
        compiler_params=pltpu.CompilerParams(
            dimension_semantics=("arbitrary",)),
    )(S, x, oh, cnt, u, w2a, w2b, b2n1, n2a, n2b, n2c, b1n2, n2w2, b2n2)


def _glob_body(u, xms, gc, ga, gb, b1g, gw2, b2g, wd, b1e, uo, po):
    xm = xms[...] / jnp.maximum(gc[...], 1.0)
    t = jnp.dot(u[...], ga[...], preferred_element_type=_f32)
    t += jnp.dot(xm, gb[...], preferred_element_type=_f32) + b1g[...]
    t = jnp.maximum(t, 0.0)
    un = jnp.dot(t, gw2[...], preferred_element_type=_f32) + b2g[...]
    uo[...] = un
    po[...] = jnp.dot(un, wd[...], preferred_element_type=_f32) + b1e[...]


def _glob_call(u, xms, gc, ga, gb, b1g, gw2, b2g, wd, b1e):
    wspec = lambda r, c: pl.BlockSpec((r, c), lambda i: (0, 0))
    return pl.pallas_call(
        _glob_body,
        grid=(1,),
        in_specs=[
            wspec(G, FG), wspec(G, F), wspec(G, 1),
            wspec(FG, H), wspec(F, H), wspec(1, H),
            wspec(H, FG), wspec(1, FG),
            wspec(FG, H), wspec(1, H),
        ],
        out_specs=[
            pl.BlockSpec((G, FG), lambda i: (0, 0)),
            pl.BlockSpec((G, H), lambda i: (0, 0)),
        ],
        out_shape=[
            jax.ShapeDtypeStruct((G, FG), _f32),
            jax.ShapeDtypeStruct((G, H), _f32),
        ],
        compiler_params=pltpu.CompilerParams(
            dimension_semantics=("arbitrary",)),
    )(u, xms, gc, ga, gb, b1g, gw2, b2g, wd, b1e)


def kernel(x, edge_index, edge_attr, u, batch,
           edge_w1, edge_b1, edge_w2, edge_b2,
           node1_w1, node1_b1, node1_w2, node1_b2,
           node2_w1, node2_b1, node2_w2, node2_b2,
           glob_w1, glob_b1, glob_w2, glob_b2):
    row = edge_index[0].astype(_i32)
    col = edge_index[1].astype(_i32)
    row2 = jnp.pad(row, (0, GPAD * CH - E),
                   constant_values=N).reshape(GPAD, CH)
    col2 = jnp.pad(col, (0, GPAD * CH - E),
                   constant_values=N).reshape(GPAD, CH)
    xt = jnp.pad(x, ((0, NPAD - N), (0, 0)))
    b2 = jnp.pad(batch.astype(_i32), (0, NPAD - N),
                 constant_values=G).reshape(NPAD, 1)
    ea = jnp.pad(edge_attr, ((0, EP - E), (0, 0)))

    wxx = edge_w1[:2 * F]
    wc = edge_w1[2 * F:2 * F + FE]
    wd = edge_w1[2 * F + FE:]
    b1e = edge_b1.reshape(1, H)
    b2e = edge_b2.reshape(1, FE)
    wna = node1_w1[:F]
    wnb = node1_w1[F:]
    b1n = node1_b1.reshape(1, H)
    w2a = node1_w2[:F]
    w2b = node1_w2[F:]
    b2n1 = node1_b2.reshape(1, H)
    n2a = node2_w1[:F]
    n2b = node2_w1[F:F + H]
    n2c = node2_w1[F + H:]
    b1n2 = node2_b1.reshape(1, H)
    b2n2 = node2_b2.reshape(1, F)
    ga = glob_w1[:FG]
    gb = glob_w1[FG:]
    b1g = glob_b1.reshape(1, H)
    b2g = glob_b2.reshape(1, FG)

    bat = jnp.pad(batch.astype(_i32), (0, NPAD - N), constant_values=0)
    b128 = jnp.broadcast_to(bat[:, None], (NPAD, F))
    cntp = _count_call(col2)
    gw = _gidx_call(row2, b128)
    gi = _compact_call(gw)
    oh, pmat, cnt, gc = _prep_call(b2, cntp, u, wd, b1e)

    for _ in range(3):
        xr, xc = _gather_call(xt, row2, col2)
        ea, h3 = _edge_call(xr, xc, gi, ea, wxx, wc, pmat, edge_w2, b2e,
                            wna, wnb, b1n)
        S = _scatter_call(h3, col2)
        xt, xms = _node_call(S, xt, oh, cnt, u, w2a, w2b, b2n1,
                             n2a, n2b, n2c, b1n2, node2_w2, b2n2)
        u, pmat = _glob_call(u, xms, gc, ga, gb, b1g, glob_w2, b2g, wd, b1e)

    return xt[:N], ea[:E], u


# speedup vs baseline: 1.0863x; 1.0863x over previous
import functools

import jax
import jax.numpy as jnp
from jax import lax
from jax.experimental import pallas as pl
from jax.experimental.pallas import tpu as pltpu
from jax.experimental.pallas import tpu_sc as plsc

N = 10000
E = 160000
F = 128
FE = 16
FG = 16
H = 256
G = 64

NC = 2
NS = 16
NW = NC * NS
CH = 128
EP = 163840
NCH = EP // (NW * CH)
SCH = EP // (NS * CH)
NPAD = 10240
NROW = NPAD // NS
BE = 512
BN = 1024

_f32 = jnp.float32
_i32 = jnp.int32


def _mesh():
    return plsc.VectorSubcoreMesh(core_axis_name="c", subcore_axis_name="s",
                                  num_cores=NC, num_subcores=NS)


def _count_body(col2, cntp, coli, buf, obuf, acc):
    c = lax.axis_index("c")
    s = lax.axis_index("s")
    w = s * NC + c
    zero16 = jnp.zeros((16,), _f32)
    one16 = jnp.ones((16,), _f32)

    def zb(i, carry):
        for j in range(F // 16):
            buf[i, pl.ds(j * 16, 16)] = zero16
            obuf[i, pl.ds(j * 16, 16)] = one16
        return carry

    lax.fori_loop(0, CH, zb, 0)

    def zc(k, carry):
        pltpu.sync_copy(buf, acc.at[pl.ds(s * NROW + k * CH, CH)])
        return carry

    lax.fori_loop(0, NROW // CH, zc, 0)
    plsc.subcore_barrier()
    pltpu.sync_copy(col2.at[pl.ds(w * NCH, NCH)], coli)

    def step(i, carry):
        pltpu.sync_copy(obuf, acc.at[coli.at[i]], add=True)
        return carry

    lax.fori_loop(0, NCH, step, 0)
    plsc.subcore_barrier()
    pltpu.sync_copy(acc.at[pl.ds(s * NROW, NROW)],
                    cntp.at[c, pl.ds(s * NROW, NROW)])


def _count_call(col2):
    k = pl.kernel(
        _count_body,
        out_type=jax.ShapeDtypeStruct((NC, NPAD, F), _f32),
        mesh=_mesh(),
        scratch_types=[
            pltpu.VMEM((NCH, CH), _i32),
            pltpu.VMEM((CH, F), _f32),
            pltpu.VMEM((CH, F), _f32),
            pltpu.VMEM_SHARED((NPAD, F), _f32),
        ],
    )
    return k(col2)


def _gidx_body(row2, b128, gw, rowi, bufg, s1):
    c = lax.axis_index("c")
    s = lax.axis_index("s")
    w = s * NC + c
    pltpu.sync_copy(row2.at[pl.ds(w * NCH, NCH)], rowi)

    def step(i, carry):
        ebase = (w * NCH + i) * CH
        pltpu.async_copy(b128.at[rowi.at[i]], bufg, s1).wait()
        pltpu.sync_copy(bufg, gw.at[pl.ds(ebase, CH)])
        return carry

    lax.fori_loop(0, NCH, step, 0)


def _gidx_call(row2, b128):
    k = pl.kernel(
        _gidx_body,
        out_type=jax.ShapeDtypeStruct((EP, F), _i32),
        mesh=_mesh(),
        scratch_types=[
            pltpu.VMEM((NCH, CH), _i32),
            pltpu.VMEM((CH, F), _i32),
            pltpu.SemaphoreType.DMA,
        ],
    )
    return k(row2, b128)


def _compact_body(gw, gi_o):
    gi_o[...] = gw[...][:, :1]


def _compact_call(gw):
    return pl.pallas_call(
        _compact_body,
        grid=(EP // BE,),
        in_specs=[pl.BlockSpec((BE, F), lambda i: (i, 0))],
        out_specs=pl.BlockSpec((BE, 1), lambda i: (i, 0)),
        out_shape=jax.ShapeDtypeStruct((EP, 1), _i32),
        compiler_params=pltpu.CompilerParams(
            dimension_semantics=("arbitrary",)),
    )(gw)


GA = 72
GB = 8
GPAD = 1344


def _gather_body(xtab, row2, col2, xr_o, xc_o,
                 rowi, coli, bxr0, bxc0, bxr1, bxc1, sr0, sc0, sr1, sc1):
    c = lax.axis_index("c")
    s = lax.axis_index("s")
    nch = jnp.where(c == 0, GA, GB)
    cbase = jnp.where(c == 0, s * GA, 16 * GA + s * GB)
    pltpu.sync_copy(row2.at[pl.ds(cbase, GA)], rowi)
    pltpu.sync_copy(col2.at[pl.ds(cbase, GA)], coli)

    def start(l, br, bc, svr, svc):
        pltpu.async_copy(xtab.at[rowi.at[l]], br, svr)
        pltpu.async_copy(xtab.at[coli.at[l]], bc, svc)

    def finish(l, br, bc, svr, svc):
        pltpu.make_async_copy(xtab.at[rowi.at[l]], br, svr).wait()
        pltpu.make_async_copy(xtab.at[coli.at[l]], bc, svc).wait()
        base = (cbase + l) * CH
        pltpu.sync_copy(br, xr_o.at[pl.ds(base, CH)])
        pltpu.sync_copy(bc, xc_o.at[pl.ds(base, CH)])

    start(0, bxr0, bxc0, sr0, sc0)
    start(1, bxr1, bxc1, sr1, sc1)

    def step(k, carry):
        i0 = 2 * k
        finish(i0, bxr0, bxc0, sr0, sc0)
        start(i0 + 2, bxr0, bxc0, sr0, sc0)
        finish(i0 + 1, bxr1, bxc1, sr1, sc1)
        start(i0 + 3, bxr1, bxc1, sr1, sc1)
        return carry

    lax.fori_loop(0, nch // 2 - 1, step, 0)
    finish(nch - 2, bxr0, bxc0, sr0, sc0)
    finish(nch - 1, bxr1, bxc1, sr1, sc1)


def _gather_call(xtab, row2p, col2p):
    k = pl.kernel(
        _gather_body,
        out_type=[
            jax.ShapeDtypeStruct((EP, F), _f32),
            jax.ShapeDtypeStruct((EP, F), _f32),
        ],
        mesh=_mesh(),
        scratch_types=[
            pltpu.VMEM((GA, CH), _i32),
            pltpu.VMEM((GA, CH), _i32),
            pltpu.VMEM((CH, F), _f32),
            pltpu.VMEM((CH, F), _f32),
            pltpu.VMEM((CH, F), _f32),
            pltpu.VMEM((CH, F), _f32),
            pltpu.SemaphoreType.DMA,
            pltpu.SemaphoreType.DMA,
            pltpu.SemaphoreType.DMA,
            pltpu.SemaphoreType.DMA,
        ],
    )
    return k(xtab, row2p, col2p)


def _scatter_body(h3, col2, s_out, coli, buf, acc, s1):
    c = lax.axis_index("c")
    s = lax.axis_index("s")
    zero16 = jnp.zeros((16,), _f32)

    def zb(i, carry):
        for j in range(F // 16):
            buf[i, pl.ds(j * 16, 16)] = zero16
        return carry

    lax.fori_loop(0, CH, zb, 0)

    def zc(k, carry):
        pltpu.sync_copy(buf, acc.at[pl.ds(s * NROW + k * CH, CH)])
        return carry

    lax.fori_loop(0, NROW // CH, zc, 0)
    plsc.subcore_barrier()

    pltpu.sync_copy(col2.at[pl.ds(s * SCH, SCH)], coli)

    def step(i, carry):
        gbase = (s * SCH + i) * CH
        pltpu.async_copy(h3.at[c, pl.ds(gbase, CH)], buf, s1).wait()
        pltpu.sync_copy(buf, acc.at[coli.at[i]], add=True)
        return carry

    lax.fori_loop(0, SCH, step, 0)
    plsc.subcore_barrier()
    pltpu.sync_copy(acc.at[pl.ds(s * NROW, NROW)],
                    s_out.at[c, pl.ds(s * NROW, NROW)])


def _scatter_call(h3, col2):
    k = pl.kernel(
        _scatter_body,
        out_type=jax.ShapeDtypeStruct((NC, NPAD, F), _f32),
        mesh=_mesh(),
        scratch_types=[
            pltpu.VMEM((SCH, CH), _i32),
            pltpu.VMEM((CH, F), _f32),
            pltpu.VMEM_SHARED((NPAD, F), _f32),
            pltpu.SemaphoreType.DMA,
        ],
    )
    return k(h3, col2)


def _prep_body(b2, cntp, u, wd, b1e, oh_o, p_o, cnt_o, gc_o):
    oh = (b2[...] == lax.broadcasted_iota(_i32, (1, G), 1)).astype(_f32)
    oh_o[...] = oh
    p_o[...] = jnp.dot(u[...], wd[...], preferred_element_type=_f32) + b1e[...]
    cnt_o[...] = cntp[0][:, :1] + cntp[1][:, :1]

    @pl.when(pl.program_id(0) == 0)
    def _():
        gc_o[...] = jnp.zeros_like(gc_o)

    gc_o[...] += lax.dot_general(oh, jnp.ones((BN, 1), _f32),
                                 (((0,), (0,)), ((), ())),
                                 preferred_element_type=_f32)


def _prep_call(b2, cntp, u, wd, b1e):
    grid = (NPAD // BN,)
    return pl.pallas_call(
        _prep_body,
        grid=grid,
        in_specs=[
            pl.BlockSpec((BN, 1), lambda i: (i, 0)),
            pl.BlockSpec((NC, BN, F), lambda i: (0, i, 0)),
            pl.BlockSpec((G, FG), lambda i: (0, 0)),
            pl.BlockSpec((FG, H), lambda i: (0, 0)),
            pl.BlockSpec((1, H), lambda i: (0, 0)),
        ],
        out_specs=[
            pl.BlockSpec((BN, G), lambda i: (i, 0)),
            pl.BlockSpec((G, H), lambda i: (0, 0)),
            pl.BlockSpec((BN, 1), lambda i: (i, 0)),
            pl.BlockSpec((G, 1), lambda i: (0, 0)),
        ],
        out_shape=[
            jax.ShapeDtypeStruct((NPAD, G), _f32),
            jax.ShapeDtypeStruct((G, H), _f32),
            jax.ShapeDtypeStruct((NPAD, 1), _f32),
            jax.ShapeDtypeStruct((G, 1), _f32),
        ],
        compiler_params=pltpu.CompilerParams(
            dimension_semantics=("arbitrary",)),
    )(b2, cntp, u, wd, b1e)


def _edge_body(xr, xc, gi, ea, wxx, wc, pmat, w2e, b2e, wna, wnb, b1n,
               ea_o, h3_o):
    xx = jnp.concatenate([xr[...], xc[...]], axis=1)
    ohe = (gi[...] == lax.broadcasted_iota(_i32, (1, G), 1)).astype(_f32)
    h = jnp.dot(xx, wxx[...], preferred_element_type=_f32)
    h += jnp.dot(ea[...], wc[...], preferred_element_type=_f32)
    h += jnp.dot(ohe, pmat[...], preferred_element_type=_f32)
    h = jnp.maximum(h, 0.0)
    ean = jnp.dot(h, w2e[...], preferred_element_type=_f32) + b2e[...]
    hn = jnp.dot(xr[...], wna[...], preferred_element_type=_f32)
    hn += jnp.dot(ean, wnb[...], preferred_element_type=_f32) + b1n[...]
    hn = jnp.maximum(hn, 0.0)
    m = pl.program_id(0) * BE + lax.broadcasted_iota(_i32, (BE, 1), 0) < E
    ea_o[...] = jnp.where(m, ean, 0.0)
    h3_o[0] = jnp.where(m, hn[:, :F], 0.0)
    h3_o[1] = jnp.where(m, hn[:, F:], 0.0)


def _edge_call(xr, xc, gi, ea, wxx, wc, pmat, w2e, b2e, wna, wnb, b1n):
    grid = (EP // BE,)
    wspec = lambda r, c: pl.BlockSpec((r, c), lambda i: (0, 0))
    return pl.pallas_call(
        _edge_body,
        grid=grid,
        in_specs=[
            pl.BlockSpec((BE, F), lambda i: (i, 0)),
            pl.BlockSpec((BE, F), lambda i: (i, 0)),
            pl.BlockSpec((BE, 1), lambda i: (i, 0)),
            pl.BlockSpec((BE, FE), lambda i: (i, 0)),
            wspec(2 * F, H), wspec(FE, H), wspec(G, H),
            wspec(H, FE), wspec(1, FE),
            wspec(F, H), wspec(FE, H), wspec(1, H),
        ],
        out_specs=[
            pl.BlockSpec((BE, FE), lambda i: (i, 0)),
            pl.BlockSpec((NC, BE, F), lambda i: (0, i, 0)),
        ],
        out_shape=[
            jax.ShapeDtypeStruct((EP, FE), _f32),
            jax.ShapeDtypeStruct((NC, EP, F), _f32),
        ],
        compiler_params=pltpu.CompilerParams(
            dimension_semantics=("arbitrary",)),
    )(xr, xc, gi, ea, wxx, wc, pmat, w2e, b2e, wna, wnb, b1n)


def _node_body(S, x, oh, cnt, u, w2a, w2b, b2n1, n2a, n2b, n2c, b1n2,
               n2w2, b2n2, xo, xmo):
    cnt_ = cnt[...]
    agg = jnp.dot(S[0], w2a[...], preferred_element_type=_f32)
    agg += jnp.dot(S[1], w2b[...], preferred_element_type=_f32)
    agg = (agg + cnt_ * b2n1[...]) / jnp.maximum(cnt_, 1.0)
    ub = jnp.dot(oh[...], u[...], preferred_element_type=_f32)
    t = jnp.dot(x[...], n2a[...], preferred_element_type=_f32)
    t += jnp.dot(agg, n2b[...], preferred_element_type=_f32)
    t += jnp.dot(ub, n2c[...], preferred_element_type=_f32) + b1n2[...]
    t = jnp.maximum(t, 0.0)
    xn = jnp.dot(t, n2w2[...], preferred_element_type=_f32) + b2n2[...]
    xo[...] = xn

    @pl.when(pl.program_id(0) == 0)
    def _():
        xmo[...] = jnp.zeros_like(xmo)

    xmo[...] += lax.dot_general(oh[...], xn, (((0,), (0,)), ((), ())),
                                preferred_element_type=_f32)


def _node_call(S, x, oh, cnt, u, w2a, w2b, b2n1, n2a, n2b, n2c, b1n2,
               n2w2, b2n2):
    grid = (NPAD // BN,)
    wspec = lambda r, c: pl.BlockSpec((r, c), lambda i: (0, 0))
    return pl.pallas_call(
        _node_body,
        grid=grid,
        in_specs=[
            pl.BlockSpec((NC, BN, F), lambda i: (0, i, 0)),
            pl.BlockSpec((BN, F), lambda i: (i, 0)),
            pl.BlockSpec((BN, G), lambda i: (i, 0)),
            pl.BlockSpec((BN, 1), lambda i: (i, 0)),
            wspec(G, FG),
            wspec(F, H), wspec(F, H), wspec(1, H),
            wspec(F, H), wspec(H, H), wspec(FG, H), wspec(1, H),
            wspec(H, F), wspec(1, F),
        ],
        out_specs=[
            pl.BlockSpec((BN, F), lambda i: (i, 0)),
            pl.BlockSpec((G, F), lambda i: (0, 0)),
        ],
        out_shape=[
            jax.ShapeDtypeStruct((NPAD, F), _f32),
            jax.ShapeDtypeStruct((G, F), _f32),
        ],
        compiler_params=pltpu.CompilerParams(
            dimension_semantics=("arbitrary",)),
    )(S, x, oh, cnt, u, w2a, w2b, b2n1, n2a, n2b, n2c, b1n2, n2w2, b2n2)


def _glob_body(u, xms, gc, ga, gb, b1g, gw2, b2g, wd, b1e, uo, po):
    xm = xms[...] / jnp.maximum(gc[...], 1.0)
    t = jnp.dot(u[...], ga[...], preferred_element_type=_f32)
    t += jnp.dot(xm, gb[...], preferred_element_type=_f32) + b1g[...]
    t = jnp.maximum(t, 0.0)
    un = jnp.dot(t, gw2[...], preferred_element_type=_f32) + b2g[...]
    uo[...] = un
    po[...] = jnp.dot(un, wd[...], preferred_element_type=_f32) + b1e[...]


def _glob_call(u, xms, gc, ga, gb, b1g, gw2, b2g, wd, b1e):
    wspec = lambda r, c: pl.BlockSpec((r, c), lambda i: (0, 0))
    return pl.pallas_call(
        _glob_body,
        grid=(1,),
        in_specs=[
            wspec(G, FG), wspec(G, F), wspec(G, 1),
            wspec(FG, H), wspec(F, H), wspec(1, H),
            wspec(H, FG), wspec(1, FG),
            wspec(FG, H), wspec(1, H),
        ],
        out_specs=[
            pl.BlockSpec((G, FG), lambda i: (0, 0)),
            pl.BlockSpec((G, H), lambda i: (0, 0)),
        ],
        out_shape=[
            jax.ShapeDtypeStruct((G, FG), _f32),
            jax.ShapeDtypeStruct((G, H), _f32),
        ],
        compiler_params=pltpu.CompilerParams(
            dimension_semantics=("arbitrary",)),
    )(u, xms, gc, ga, gb, b1g, gw2, b2g, wd, b1e)


def kernel(x, edge_index, edge_attr, u, batch,
           edge_w1, edge_b1, edge_w2, edge_b2,
           node1_w1, node1_b1, node1_w2, node1_b2,
           node2_w1, node2_b1, node2_w2, node2_b2,
           glob_w1, glob_b1, glob_w2, glob_b2):
    row = edge_index[0].astype(_i32)
    col = edge_index[1].astype(_i32)
    row2 = jnp.pad(row, (0, GPAD * CH - E),
                   constant_values=N).reshape(GPAD, CH)
    col2 = jnp.pad(col, (0, GPAD * CH - E),
                   constant_values=N).reshape(GPAD, CH)
    xt = jnp.pad(x, ((0, NPAD - N), (0, 0)))
    b2 = jnp.pad(batch.astype(_i32), (0, NPAD - N),
                 constant_values=G).reshape(NPAD, 1)
    ea = jnp.pad(edge_attr, ((0, EP - E), (0, 0)))

    wxx = edge_w1[:2 * F]
    wc = edge_w1[2 * F:2 * F + FE]
    wd = edge_w1[2 * F + FE:]
    b1e = edge_b1.reshape(1, H)
    b2e = edge_b2.reshape(1, FE)
    wna = node1_w1[:F]
    wnb = node1_w1[F:]
    b1n = node1_b1.reshape(1, H)
    w2a = node1_w2[:F]
    w2b = node1_w2[F:]
    b2n1 = node1_b2.reshape(1, H)
    n2a = node2_w1[:F]
    n2b = node2_w1[F:F + H]
    n2c = node2_w1[F + H:]
    b1n2 = node2_b1.reshape(1, H)
    b2n2 = node2_b2.reshape(1, F)
    ga = glob_w1[:FG]
    gb = glob_w1[FG:]
    b1g = glob_b1.reshape(1, H)
    b2g = glob_b2.reshape(1, FG)

    bat = jnp.pad(batch.astype(_i32), (0, NPAD - N), constant_values=0)
    b128 = jnp.broadcast_to(bat[:, None], (NPAD, F))
    cntp = _count_call(col2)
    gw = _gidx_call(row2, b128)
    gi = _compact_call(gw)
    oh, pmat, cnt, gc = _prep_call(b2, cntp, u, wd, b1e)

    for _ in range(3):
        xr, xc = _gather_call(xt, row2, col2)
        ea, h3 = _edge_call(xr, xc, gi, ea, wxx, wc, pmat, edge_w2, b2e,
                            wna, wnb, b1n)
        S = _scatter_call(h3, col2)
        xt, xms = _node_call(S, xt, oh, cnt, u, w2a, w2b, b2n1,
                             n2a, n2b, n2c, b1n2, node2_w2, b2n2)
        u, pmat = _glob_call(u, xms, gc, ga, gb, b1g, glob_w2, b2g, wd, b1e)

    return xt[:N], ea[:E], u


# speedup vs baseline: 1.1877x; 1.0933x over previous
import functools

import jax
import jax.numpy as jnp
from jax import lax
from jax.experimental import pallas as pl
from jax.experimental.pallas import tpu as pltpu
from jax.experimental.pallas import tpu_sc as plsc

N = 10000
E = 160000
F = 128
FE = 16
FG = 16
H = 256
G = 64

NC = 2
NS = 16
NW = NC * NS
CH = 128
EP = 163840
NCH = EP // (NW * CH)
SCH = EP // (NS * CH)
NPAD = 10240
NROW = NPAD // NS
BE = 1024
BN = 1024

_f32 = jnp.float32
_i32 = jnp.int32


def _mesh():
    return plsc.VectorSubcoreMesh(core_axis_name="c", subcore_axis_name="s",
                                  num_cores=NC, num_subcores=NS)


def _count_body(col2, cntp, coli, buf, obuf, acc):
    c = lax.axis_index("c")
    s = lax.axis_index("s")
    w = s * NC + c
    zero16 = jnp.zeros((16,), _f32)
    one16 = jnp.ones((16,), _f32)

    def zb(i, carry):
        for j in range(F // 16):
            buf[i, pl.ds(j * 16, 16)] = zero16
            obuf[i, pl.ds(j * 16, 16)] = one16
        return carry

    lax.fori_loop(0, CH, zb, 0)

    def zc(k, carry):
        pltpu.sync_copy(buf, acc.at[pl.ds(s * NROW + k * CH, CH)])
        return carry

    lax.fori_loop(0, NROW // CH, zc, 0)
    plsc.subcore_barrier()
    pltpu.sync_copy(col2.at[pl.ds(w * NCH, NCH)], coli)

    def step(i, carry):
        pltpu.sync_copy(obuf, acc.at[coli.at[i]], add=True)
        return carry

    lax.fori_loop(0, NCH, step, 0)
    plsc.subcore_barrier()
    pltpu.sync_copy(acc.at[pl.ds(s * NROW, NROW)],
                    cntp.at[c, pl.ds(s * NROW, NROW)])


def _count_call(col2):
    k = pl.kernel(
        _count_body,
        out_type=jax.ShapeDtypeStruct((NC, NPAD, F), _f32),
        mesh=_mesh(),
        scratch_types=[
            pltpu.VMEM((NCH, CH), _i32),
            pltpu.VMEM((CH, F), _f32),
            pltpu.VMEM((CH, F), _f32),
            pltpu.VMEM_SHARED((NPAD, F), _f32),
        ],
    )
    return k(col2)


def _gidx_body(row2, b128, gw, rowi, bufg, s1):
    c = lax.axis_index("c")
    s = lax.axis_index("s")
    w = s * NC + c
    pltpu.sync_copy(row2.at[pl.ds(w * NCH, NCH)], rowi)

    def step(i, carry):
        ebase = (w * NCH + i) * CH
        pltpu.async_copy(b128.at[rowi.at[i]], bufg, s1).wait()
        pltpu.sync_copy(bufg, gw.at[pl.ds(ebase, CH)])
        return carry

    lax.fori_loop(0, NCH, step, 0)


def _gidx_call(row2, b128):
    k = pl.kernel(
        _gidx_body,
        out_type=jax.ShapeDtypeStruct((EP, F), _i32),
        mesh=_mesh(),
        scratch_types=[
            pltpu.VMEM((NCH, CH), _i32),
            pltpu.VMEM((CH, F), _i32),
            pltpu.SemaphoreType.DMA,
        ],
    )
    return k(row2, b128)


def _compact_body(gw, gi_o):
    gi_o[...] = gw[...][:, :1]


def _compact_call(gw):
    return pl.pallas_call(
        _compact_body,
        grid=(EP // BE,),
        in_specs=[pl.BlockSpec((BE, F), lambda i: (i, 0))],
        out_specs=pl.BlockSpec((BE, 1), lambda i: (i, 0)),
        out_shape=jax.ShapeDtypeStruct((EP, 1), _i32),
        compiler_params=pltpu.CompilerParams(
            dimension_semantics=("arbitrary",)),
    )(gw)


GA = 72
GB = 8
GPAD = 1344


def _gather_body(xtab, row2, col2, xr_o, xc_o,
                 rowi, coli, bxr0, bxc0, bxr1, bxc1, sr0, sc0, sr1, sc1):
    c = lax.axis_index("c")
    s = lax.axis_index("s")
    nch = jnp.where(c == 0, GA, GB)
    cbase = jnp.where(c == 0, s * GA, 16 * GA + s * GB)
    pltpu.sync_copy(row2.at[pl.ds(cbase, GA)], rowi)
    pltpu.sync_copy(col2.at[pl.ds(cbase, GA)], coli)

    def start(l, br, bc, svr, svc):
        pltpu.async_copy(xtab.at[rowi.at[l]], br, svr)
        pltpu.async_copy(xtab.at[coli.at[l]], bc, svc)

    def finish(l, br, bc, svr, svc):
        pltpu.make_async_copy(xtab.at[rowi.at[l]], br, svr).wait()
        pltpu.make_async_copy(xtab.at[coli.at[l]], bc, svc).wait()
        base = (cbase + l) * CH
        pltpu.sync_copy(br, xr_o.at[pl.ds(base, CH)])
        pltpu.sync_copy(bc, xc_o.at[pl.ds(base, CH)])

    start(0, bxr0, bxc0, sr0, sc0)
    start(1, bxr1, bxc1, sr1, sc1)

    def step(k, carry):
        i0 = 2 * k
        finish(i0, bxr0, bxc0, sr0, sc0)
        start(i0 + 2, bxr0, bxc0, sr0, sc0)
        finish(i0 + 1, bxr1, bxc1, sr1, sc1)
        start(i0 + 3, bxr1, bxc1, sr1, sc1)
        return carry

    lax.fori_loop(0, nch // 2 - 1, step, 0)
    finish(nch - 2, bxr0, bxc0, sr0, sc0)
    finish(nch - 1, bxr1, bxc1, sr1, sc1)


def _gather_call(xtab, row2p, col2p):
    k = pl.kernel(
        _gather_body,
        out_type=[
            jax.ShapeDtypeStruct((EP, F), _f32),
            jax.ShapeDtypeStruct((EP, F), _f32),
        ],
        mesh=_mesh(),
        scratch_types=[
            pltpu.VMEM((GA, CH), _i32),
            pltpu.VMEM((GA, CH), _i32),
            pltpu.VMEM((CH, F), _f32),
            pltpu.VMEM((CH, F), _f32),
            pltpu.VMEM((CH, F), _f32),
            pltpu.VMEM((CH, F), _f32),
            pltpu.SemaphoreType.DMA,
            pltpu.SemaphoreType.DMA,
            pltpu.SemaphoreType.DMA,
            pltpu.SemaphoreType.DMA,
        ],
    )
    return k(xtab, row2p, col2p)


def _scatter_body(h3, col2, s_out, coli, buf, acc, s1):
    c = lax.axis_index("c")
    s = lax.axis_index("s")
    zero16 = jnp.zeros((16,), _f32)

    def zb(i, carry):
        for j in range(F // 16):
            buf[i, pl.ds(j * 16, 16)] = zero16
        return carry

    lax.fori_loop(0, CH, zb, 0)

    def zc(k, carry):
        pltpu.sync_copy(buf, acc.at[pl.ds(s * NROW + k * CH, CH)])
        return carry

    lax.fori_loop(0, NROW // CH, zc, 0)
    plsc.subcore_barrier()

    pltpu.sync_copy(col2.at[pl.ds(s * SCH, SCH)], coli)

    def step(i, carry):
        gbase = (s * SCH + i) * CH
        pltpu.async_copy(h3.at[c, pl.ds(gbase, CH)], buf, s1).wait()
        pltpu.sync_copy(buf, acc.at[coli.at[i]], add=True)
        return carry

    lax.fori_loop(0, SCH, step, 0)
    plsc.subcore_barrier()
    pltpu.sync_copy(acc.at[pl.ds(s * NROW, NROW)],
                    s_out.at[c, pl.ds(s * NROW, NROW)])


def _scatter_call(h3, col2):
    k = pl.kernel(
        _scatter_body,
        out_type=jax.ShapeDtypeStruct((NC, NPAD, F), _f32),
        mesh=_mesh(),
        scratch_types=[
            pltpu.VMEM((SCH, CH), _i32),
            pltpu.VMEM((CH, F), _f32),
            pltpu.VMEM_SHARED((NPAD, F), _f32),
            pltpu.SemaphoreType.DMA,
        ],
    )
    return k(h3, col2)


def _prep_body(b2, cntp, u, wd, b1e, oh_o, p_o, cnt_o, gc_o):
    oh = (b2[...] == lax.broadcasted_iota(_i32, (1, G), 1)).astype(_f32)
    oh_o[...] = oh
    p_o[...] = jnp.dot(u[...], wd[...], preferred_element_type=_f32) + b1e[...]
    cnt_o[...] = cntp[0][:, :1] + cntp[1][:, :1]

    @pl.when(pl.program_id(0) == 0)
    def _():
        gc_o[...] = jnp.zeros_like(gc_o)

    gc_o[...] += lax.dot_general(oh, jnp.ones((BN, 1), _f32),
                                 (((0,), (0,)), ((), ())),
                                 preferred_element_type=_f32)


def _prep_call(b2, cntp, u, wd, b1e):
    grid = (NPAD // BN,)
    return pl.pallas_call(
        _prep_body,
        grid=grid,
        in_specs=[
            pl.BlockSpec((BN, 1), lambda i: (i, 0)),
            pl.BlockSpec((NC, BN, F), lambda i: (0, i, 0)),
            pl.BlockSpec((G, FG), lambda i: (0, 0)),
            pl.BlockSpec((FG, H), lambda i: (0, 0)),
            pl.BlockSpec((1, H), lambda i: (0, 0)),
        ],
        out_specs=[
            pl.BlockSpec((BN, G), lambda i: (i, 0)),
            pl.BlockSpec((G, H), lambda i: (0, 0)),
            pl.BlockSpec((BN, 1), lambda i: (i, 0)),
            pl.BlockSpec((G, 1), lambda i: (0, 0)),
        ],
        out_shape=[
            jax.ShapeDtypeStruct((NPAD, G), _f32),
            jax.ShapeDtypeStruct((G, H), _f32),
            jax.ShapeDtypeStruct((NPAD, 1), _f32),
            jax.ShapeDtypeStruct((G, 1), _f32),
        ],
        compiler_params=pltpu.CompilerParams(
            dimension_semantics=("arbitrary",)),
    )(b2, cntp, u, wd, b1e)


def _edge_body(xr, xc, gi, ea, wxx, wc, pmat, w2e, b2e, wna, wnb, b1n,
               ea_o, h3_o):
    xx = jnp.concatenate([xr[...], xc[...]], axis=1)
    ohe = (gi[...] == lax.broadcasted_iota(_i32, (1, G), 1)).astype(_f32)
    h = jnp.dot(xx, wxx[...], preferred_element_type=_f32)
    h += jnp.dot(ea[...], wc[...], preferred_element_type=_f32)
    h += jnp.dot(ohe, pmat[...], preferred_element_type=_f32)
    h = jnp.maximum(h, 0.0)
    ean = jnp.dot(h, w2e[...], preferred_element_type=_f32) + b2e[...]
    hn = jnp.dot(xr[...], wna[...], preferred_element_type=_f32)
    hn += jnp.dot(ean, wnb[...], preferred_element_type=_f32) + b1n[...]
    hn = jnp.maximum(hn, 0.0)
    ea_o[...] = ean
    h3_o[0] = hn[:, :F]
    h3_o[1] = hn[:, F:]


def _edge_call(xr, xc, gi, ea, wxx, wc, pmat, w2e, b2e, wna, wnb, b1n):
    grid = (EP // BE,)
    wspec = lambda r, c: pl.BlockSpec((r, c), lambda i: (0, 0))
    return pl.pallas_call(
        _edge_body,
        grid=grid,
        in_specs=[
            pl.BlockSpec((BE, F), lambda i: (i, 0)),
            pl.BlockSpec((BE, F), lambda i: (i, 0)),
            pl.BlockSpec((BE, 1), lambda i: (i, 0)),
            pl.BlockSpec((BE, FE), lambda i: (i, 0)),
            wspec(2 * F, H), wspec(FE, H), wspec(G, H),
            wspec(H, FE), wspec(1, FE),
            wspec(F, H), wspec(FE, H), wspec(1, H),
        ],
        out_specs=[
            pl.BlockSpec((BE, FE), lambda i: (i, 0)),
            pl.BlockSpec((NC, BE, F), lambda i: (0, i, 0)),
        ],
        out_shape=[
            jax.ShapeDtypeStruct((EP, FE), _f32),
            jax.ShapeDtypeStruct((NC, EP, F), _f32),
        ],
        compiler_params=pltpu.CompilerParams(
            dimension_semantics=("arbitrary",)),
    )(xr, xc, gi, ea, wxx, wc, pmat, w2e, b2e, wna, wnb, b1n)


def _node_body(S, x, oh, cnt, u, w2a, w2b, b2n1, n2a, n2b, n2c, b1n2,
               n2w2, b2n2, xo, xmo):
    cnt_ = cnt[...]
    agg = jnp.dot(S[0], w2a[...], preferred_element_type=_f32)
    agg += jnp.dot(S[1], w2b[...], preferred_element_type=_f32)
    agg = (agg + cnt_ * b2n1[...]) / jnp.maximum(cnt_, 1.0)
    ub = jnp.dot(oh[...], u[...], preferred_element_type=_f32)
    t = jnp.dot(x[...], n2a[...], preferred_element_type=_f32)
    t += jnp.dot(agg, n2b[...], preferred_element_type=_f32)
    t += jnp.dot(ub, n2c[...], preferred_element_type=_f32) + b1n2[...]
    t = jnp.maximum(t, 0.0)
    xn = jnp.dot(t, n2w2[...], preferred_element_type=_f32) + b2n2[...]
    xo[...] = xn

    @pl.when(pl.program_id(0) == 0)
    def _():
        xmo[...] = jnp.zeros_like(xmo)

    xmo[...] += lax.dot_general(oh[...], xn, (((0,), (0,)), ((), ())),
                                preferred_element_type=_f32)


def _node_call(S, x, oh, cnt, u, w2a, w2b, b2n1, n2a, n2b, n2c, b1n2,
               n2w2, b2n2):
    grid = (NPAD // BN,)
    wspec = lambda r, c: pl.BlockSpec((r, c), lambda i: (0, 0))
    return pl.pallas_call(
        _node_body,
        grid=grid,
        in_specs=[
            pl.BlockSpec((NC, BN, F), lambda i: (0, i, 0)),
            pl.BlockSpec((BN, F), lambda i: (i, 0)),
            pl.BlockSpec((BN, G), lambda i: (i, 0)),
            pl.BlockSpec((BN, 1), lambda i: (i, 0)),
            wspec(G, FG),
            wspec(F, H), wspec(F, H), wspec(1, H),
            wspec(F, H), wspec(H, H), wspec(FG, H), wspec(1, H),
            wspec(H, F), wspec(1, F),
        ],
        out_specs=[
            pl.BlockSpec((BN, F), lambda i: (i, 0)),
            pl.BlockSpec((G, F), lambda i: (0, 0)),
        ],
        out_shape=[
            jax.ShapeDtypeStruct((NPAD, F), _f32),
            jax.ShapeDtypeStruct((G, F), _f32),
        ],
        compiler_params=pltpu.CompilerParams(
            dimension_semantics=("arbitrary",)),
    )(S, x, oh, cnt, u, w2a, w2b, b2n1, n2a, n2b, n2c, b1n2, n2w2, b2n2)


def _glob_body(u, xms, gc, ga, gb, b1g, gw2, b2g, wd, b1e, uo, po):
    xm = xms[...] / jnp.maximum(gc[...], 1.0)
    t = jnp.dot(u[...], ga[...], preferred_element_type=_f32)
    t += jnp.dot(xm, gb[...], preferred_element_type=_f32) + b1g[...]
    t = jnp.maximum(t, 0.0)
    un = jnp.dot(t, gw2[...], preferred_element_type=_f32) + b2g[...]
    uo[...] = un
    po[...] = jnp.dot(un, wd[...], preferred_element_type=_f32) + b1e[...]


def _glob_call(u, xms, gc, ga, gb, b1g, gw2, b2g, wd, b1e):
    wspec = lambda r, c: pl.BlockSpec((r, c), lambda i: (0, 0))
    return pl.pallas_call(
        _glob_body,
        grid=(1,),
        in_specs=[
            wspec(G, FG), wspec(G, F), wspec(G, 1),
            wspec(FG, H), wspec(F, H), wspec(1, H),
            wspec(H, FG), wspec(1, FG),
            wspec(FG, H), wspec(1, H),
        ],
        out_specs=[
            pl.BlockSpec((G, FG), lambda i: (0, 0)),
            pl.BlockSpec((G, H), lambda i: (0, 0)),
        ],
        out_shape=[
            jax.ShapeDtypeStruct((G, FG), _f32),
            jax.ShapeDtypeStruct((G, H), _f32),
        ],
        compiler_params=pltpu.CompilerParams(
            dimension_semantics=("arbitrary",)),
    )(u, xms, gc, ga, gb, b1g, gw2, b2g, wd, b1e)


def kernel(x, edge_index, edge_attr, u, batch,
           edge_w1, edge_b1, edge_w2, edge_b2,
           node1_w1, node1_b1, node1_w2, node1_b2,
           node2_w1, node2_b1, node2_w2, node2_b2,
           glob_w1, glob_b1, glob_w2, glob_b2):
    row = edge_index[0].astype(_i32)
    col = edge_index[1].astype(_i32)
    row2 = jnp.pad(row, (0, GPAD * CH - E),
                   constant_values=N).reshape(GPAD, CH)
    col2 = jnp.pad(col, (0, GPAD * CH - E),
                   constant_values=N).reshape(GPAD, CH)
    xt = jnp.pad(x, ((0, NPAD - N), (0, 0)))
    b2 = jnp.pad(batch.astype(_i32), (0, NPAD - N),
                 constant_values=G).reshape(NPAD, 1)
    ea = jnp.pad(edge_attr, ((0, EP - E), (0, 0)))

    wxx = edge_w1[:2 * F]
    wc = edge_w1[2 * F:2 * F + FE]
    wd = edge_w1[2 * F + FE:]
    b1e = edge_b1.reshape(1, H)
    b2e = edge_b2.reshape(1, FE)
    wna = node1_w1[:F]
    wnb = node1_w1[F:]
    b1n = node1_b1.reshape(1, H)
    w2a = node1_w2[:F]
    w2b = node1_w2[F:]
    b2n1 = node1_b2.reshape(1, H)
    n2a = node2_w1[:F]
    n2b = node2_w1[F:F + H]
    n2c = node2_w1[F + H:]
    b1n2 = node2_b1.reshape(1, H)
    b2n2 = node2_b2.reshape(1, F)
    ga = glob_w1[:FG]
    gb = glob_w1[FG:]
    b1g = glob_b1.reshape(1, H)
    b2g = glob_b2.reshape(1, FG)

    bat = jnp.pad(batch.astype(_i32), (0, NPAD - N), constant_values=0)
    b128 = jnp.broadcast_to(bat[:, None], (NPAD, F))
    cntp = _count_call(col2)
    gw = _gidx_call(row2, b128)
    gi = _compact_call(gw)
    oh, pmat, cnt, gc = _prep_call(b2, cntp, u, wd, b1e)

    for _ in range(3):
        xr, xc = _gather_call(xt, row2, col2)
        ea, h3 = _edge_call(xr, xc, gi, ea, wxx, wc, pmat, edge_w2, b2e,
                            wna, wnb, b1n)
        S = _scatter_call(h3, col2)
        xt, xms = _node_call(S, xt, oh, cnt, u, w2a, w2b, b2n1,
                             n2a, n2b, n2c, b1n2, node2_w2, b2n2)
        u, pmat = _glob_call(u, xms, gc, ga, gb, b1g, glob_w2, b2g, wd, b1e)

    return xt[:N], ea[:E], u


# speedup vs baseline: 1.2614x; 1.0621x over previous
import functools

import jax
import jax.numpy as jnp
from jax import lax
from jax.experimental import pallas as pl
from jax.experimental.pallas import tpu as pltpu
from jax.experimental.pallas import tpu_sc as plsc

N = 10000
E = 160000
F = 128
FE = 16
FG = 16
H = 256
G = 64

NC = 2
NS = 16
NW = NC * NS
CH = 128
EP = 163840
NCH = EP // (NW * CH)
SCH = EP // (NS * CH)
NPAD = 10240
NROW = NPAD // NS
BE = 1024
BN = 1024

_f32 = jnp.float32
_i32 = jnp.int32


def _mesh():
    return plsc.VectorSubcoreMesh(core_axis_name="c", subcore_axis_name="s",
                                  num_cores=NC, num_subcores=NS)


def _count_body(col2, cntp, coli, buf, obuf, acc):
    c = lax.axis_index("c")
    s = lax.axis_index("s")
    w = s * NC + c
    zero16 = jnp.zeros((16,), _f32)
    one16 = jnp.ones((16,), _f32)

    def zb(i, carry):
        for j in range(F // 16):
            buf[i, pl.ds(j * 16, 16)] = zero16
            obuf[i, pl.ds(j * 16, 16)] = one16
        return carry

    lax.fori_loop(0, CH, zb, 0)

    def zc(k, carry):
        pltpu.sync_copy(buf, acc.at[pl.ds(s * NROW + k * CH, CH)])
        return carry

    lax.fori_loop(0, NROW // CH, zc, 0)
    plsc.subcore_barrier()
    pltpu.sync_copy(col2.at[pl.ds(w * NCH, NCH)], coli)

    def step(i, carry):
        pltpu.sync_copy(obuf, acc.at[coli.at[i]], add=True)
        return carry

    lax.fori_loop(0, NCH, step, 0)
    plsc.subcore_barrier()
    pltpu.sync_copy(acc.at[pl.ds(s * NROW, NROW)],
                    cntp.at[c, pl.ds(s * NROW, NROW)])


def _count_call(col2):
    k = pl.kernel(
        _count_body,
        out_type=jax.ShapeDtypeStruct((NC, NPAD, F), _f32),
        mesh=_mesh(),
        scratch_types=[
            pltpu.VMEM((NCH, CH), _i32),
            pltpu.VMEM((CH, F), _f32),
            pltpu.VMEM((CH, F), _f32),
            pltpu.VMEM_SHARED((NPAD, F), _f32),
        ],
    )
    return k(col2)


def _gidx_body(row2, b128, gw, rowi, bufg, s1):
    c = lax.axis_index("c")
    s = lax.axis_index("s")
    w = s * NC + c
    pltpu.sync_copy(row2.at[pl.ds(w * NCH, NCH)], rowi)

    def step(i, carry):
        ebase = (w * NCH + i) * CH
        pltpu.async_copy(b128.at[rowi.at[i]], bufg, s1).wait()
        pltpu.sync_copy(bufg, gw.at[pl.ds(ebase, CH)])
        return carry

    lax.fori_loop(0, NCH, step, 0)


def _gidx_call(row2, b128):
    k = pl.kernel(
        _gidx_body,
        out_type=jax.ShapeDtypeStruct((EP, F), _i32),
        mesh=_mesh(),
        scratch_types=[
            pltpu.VMEM((NCH, CH), _i32),
            pltpu.VMEM((CH, F), _i32),
            pltpu.SemaphoreType.DMA,
        ],
    )
    return k(row2, b128)


def _compact_body(gw, gi_o):
    gi_o[...] = gw[...][:, :1]


def _compact_call(gw):
    return pl.pallas_call(
        _compact_body,
        grid=(EP // BE,),
        in_specs=[pl.BlockSpec((BE, F), lambda i: (i, 0))],
        out_specs=pl.BlockSpec((BE, 1), lambda i: (i, 0)),
        out_shape=jax.ShapeDtypeStruct((EP, 1), _i32),
        compiler_params=pltpu.CompilerParams(
            dimension_semantics=("arbitrary",)),
    )(gw)


GA = 72
GB = 8
GPAD = 1344


def _gather_body(xtab, row2, col2, xr_o, xc_o,
                 rowi, coli, bxr0, bxc0, bxr1, bxc1, sr0, sc0, sr1, sc1):
    c = lax.axis_index("c")
    s = lax.axis_index("s")
    nch = jnp.where(c == 0, GA, GB)
    cbase = jnp.where(c == 0, s * GA, 16 * GA + s * GB)
    pltpu.sync_copy(row2.at[pl.ds(cbase, GA)], rowi)
    pltpu.sync_copy(col2.at[pl.ds(cbase, GA)], coli)

    def start(l, br, bc, svr, svc):
        pltpu.async_copy(xtab.at[rowi.at[l]], br, svr)
        pltpu.async_copy(xtab.at[coli.at[l]], bc, svc)

    def finish(l, br, bc, svr, svc):
        pltpu.make_async_copy(xtab.at[rowi.at[l]], br, svr).wait()
        pltpu.make_async_copy(xtab.at[coli.at[l]], bc, svc).wait()
        base = (cbase + l) * CH
        pltpu.sync_copy(br, xr_o.at[pl.ds(base, CH)])
        pltpu.sync_copy(bc, xc_o.at[pl.ds(base, CH)])

    start(0, bxr0, bxc0, sr0, sc0)
    start(1, bxr1, bxc1, sr1, sc1)

    def step(k, carry):
        i0 = 2 * k
        finish(i0, bxr0, bxc0, sr0, sc0)
        start(i0 + 2, bxr0, bxc0, sr0, sc0)
        finish(i0 + 1, bxr1, bxc1, sr1, sc1)
        start(i0 + 3, bxr1, bxc1, sr1, sc1)
        return carry

    lax.fori_loop(0, nch // 2 - 1, step, 0)
    finish(nch - 2, bxr0, bxc0, sr0, sc0)
    finish(nch - 1, bxr1, bxc1, sr1, sc1)


def _gather_call(xtab, row2p, col2p):
    k = pl.kernel(
        _gather_body,
        out_type=[
            jax.ShapeDtypeStruct((EP, F), _f32),
            jax.ShapeDtypeStruct((EP, F), _f32),
        ],
        mesh=_mesh(),
        scratch_types=[
            pltpu.VMEM((GA, CH), _i32),
            pltpu.VMEM((GA, CH), _i32),
            pltpu.VMEM((CH, F), _f32),
            pltpu.VMEM((CH, F), _f32),
            pltpu.VMEM((CH, F), _f32),
            pltpu.VMEM((CH, F), _f32),
            pltpu.SemaphoreType.DMA,
            pltpu.SemaphoreType.DMA,
            pltpu.SemaphoreType.DMA,
            pltpu.SemaphoreType.DMA,
        ],
    )
    return k(xtab, row2p, col2p)


def _scatter_body(h3, col2, s_out, coli, buf0, buf1, acc, s0, s1):
    c = lax.axis_index("c")
    s = lax.axis_index("s")
    zero16 = jnp.zeros((16,), _f32)

    def zb(i, carry):
        for j in range(F // 16):
            buf0[i, pl.ds(j * 16, 16)] = zero16
        return carry

    lax.fori_loop(0, CH, zb, 0)

    def zc(k, carry):
        pltpu.sync_copy(buf0, acc.at[pl.ds(s * NROW + k * CH, CH)])
        return carry

    lax.fori_loop(0, NROW // CH, zc, 0)
    plsc.subcore_barrier()

    pltpu.sync_copy(col2.at[pl.ds(s * SCH, SCH)], coli)

    def start(l, b, sv):
        pltpu.async_copy(h3.at[c, pl.ds((s * SCH + l) * CH, CH)], b, sv)

    def finish(l, b, sv):
        pltpu.make_async_copy(h3.at[c, pl.ds((s * SCH + l) * CH, CH)],
                              b, sv).wait()
        pltpu.sync_copy(b, acc.at[coli.at[l]], add=True)

    start(0, buf0, s0)
    start(1, buf1, s1)

    def step(k, carry):
        i0 = 2 * k
        finish(i0, buf0, s0)
        start(i0 + 2, buf0, s0)
        finish(i0 + 1, buf1, s1)
        start(i0 + 3, buf1, s1)
        return carry

    lax.fori_loop(0, SCH // 2 - 1, step, 0)
    finish(SCH - 2, buf0, s0)
    finish(SCH - 1, buf1, s1)
    plsc.subcore_barrier()
    pltpu.sync_copy(acc.at[pl.ds(s * NROW, NROW)],
                    s_out.at[c, pl.ds(s * NROW, NROW)])


def _scatter_call(h3, col2):
    k = pl.kernel(
        _scatter_body,
        out_type=jax.ShapeDtypeStruct((NC, NPAD, F), _f32),
        mesh=_mesh(),
        scratch_types=[
            pltpu.VMEM((SCH, CH), _i32),
            pltpu.VMEM((CH, F), _f32),
            pltpu.VMEM((CH, F), _f32),
            pltpu.VMEM_SHARED((NPAD, F), _f32),
            pltpu.SemaphoreType.DMA,
            pltpu.SemaphoreType.DMA,
        ],
    )
    return k(h3, col2)


def _prep_body(b2, cntp, u, wd, b1e, oh_o, p_o, cnt_o, gc_o):
    oh = (b2[...] == lax.broadcasted_iota(_i32, (1, G), 1)).astype(_f32)
    oh_o[...] = oh
    p_o[...] = jnp.dot(u[...], wd[...], preferred_element_type=_f32) + b1e[...]
    cnt_o[...] = cntp[0][:, :1] + cntp[1][:, :1]

    @pl.when(pl.program_id(0) == 0)
    def _():
        gc_o[...] = jnp.zeros_like(gc_o)

    gc_o[...] += lax.dot_general(oh, jnp.ones((BN, 1), _f32),
                                 (((0,), (0,)), ((), ())),
                                 preferred_element_type=_f32)


def _prep_call(b2, cntp, u, wd, b1e):
    grid = (NPAD // BN,)
    return pl.pallas_call(
        _prep_body,
        grid=grid,
        in_specs=[
            pl.BlockSpec((BN, 1), lambda i: (i, 0)),
            pl.BlockSpec((NC, BN, F), lambda i: (0, i, 0)),
            pl.BlockSpec((G, FG), lambda i: (0, 0)),
            pl.BlockSpec((FG, H), lambda i: (0, 0)),
            pl.BlockSpec((1, H), lambda i: (0, 0)),
        ],
        out_specs=[
            pl.BlockSpec((BN, G), lambda i: (i, 0)),
            pl.BlockSpec((G, H), lambda i: (0, 0)),
            pl.BlockSpec((BN, 1), lambda i: (i, 0)),
            pl.BlockSpec((G, 1), lambda i: (0, 0)),
        ],
        out_shape=[
            jax.ShapeDtypeStruct((NPAD, G), _f32),
            jax.ShapeDtypeStruct((G, H), _f32),
            jax.ShapeDtypeStruct((NPAD, 1), _f32),
            jax.ShapeDtypeStruct((G, 1), _f32),
        ],
        compiler_params=pltpu.CompilerParams(
            dimension_semantics=("arbitrary",)),
    )(b2, cntp, u, wd, b1e)


def _edge_body(xr, xc, gi, ea, wxx, wc, pmat, w2e, b2e, wna, wnb, b1n,
               ea_o, h3_o):
    xx = jnp.concatenate([xr[...], xc[...]], axis=1)
    ohe = (gi[...] == lax.broadcasted_iota(_i32, (1, G), 1)).astype(_f32)
    h = jnp.dot(xx, wxx[...], preferred_element_type=_f32)
    h += jnp.dot(ea[...], wc[...], preferred_element_type=_f32)
    h += jnp.dot(ohe, pmat[...], preferred_element_type=_f32)
    h = jnp.maximum(h, 0.0)
    ean = jnp.dot(h, w2e[...], preferred_element_type=_f32) + b2e[...]
    hn = jnp.dot(xr[...], wna[...], preferred_element_type=_f32)
    hn += jnp.dot(ean, wnb[...], preferred_element_type=_f32) + b1n[...]
    hn = jnp.maximum(hn, 0.0)
    ea_o[...] = ean
    h3_o[0] = hn[:, :F]
    h3_o[1] = hn[:, F:]


def _edge_call(xr, xc, gi, ea, wxx, wc, pmat, w2e, b2e, wna, wnb, b1n):
    grid = (EP // BE,)
    wspec = lambda r, c: pl.BlockSpec((r, c), lambda i: (0, 0))
    return pl.pallas_call(
        _edge_body,
        grid=grid,
        in_specs=[
            pl.BlockSpec((BE, F), lambda i: (i, 0)),
            pl.BlockSpec((BE, F), lambda i: (i, 0)),
            pl.BlockSpec((BE, 1), lambda i: (i, 0)),
            pl.BlockSpec((BE, FE), lambda i: (i, 0)),
            wspec(2 * F, H), wspec(FE, H), wspec(G, H),
            wspec(H, FE), wspec(1, FE),
            wspec(F, H), wspec(FE, H), wspec(1, H),
        ],
        out_specs=[
            pl.BlockSpec((BE, FE), lambda i: (i, 0)),
            pl.BlockSpec((NC, BE, F), lambda i: (0, i, 0)),
        ],
        out_shape=[
            jax.ShapeDtypeStruct((EP, FE), _f32),
            jax.ShapeDtypeStruct((NC, EP, F), _f32),
        ],
        compiler_params=pltpu.CompilerParams(
            dimension_semantics=("arbitrary",)),
    )(xr, xc, gi, ea, wxx, wc, pmat, w2e, b2e, wna, wnb, b1n)


def _node_body(S, x, oh, cnt, u, w2a, w2b, b2n1, n2a, n2b, n2c, b1n2,
               n2w2, b2n2, xo, xmo):
    cnt_ = cnt[...]
    agg = jnp.dot(S[0], w2a[...], preferred_element_type=_f32)
    agg += jnp.dot(S[1], w2b[...], preferred_element_type=_f32)
    agg = (agg + cnt_ * b2n1[...]) / jnp.maximum(cnt_, 1.0)
    ub = jnp.dot(oh[...], u[...], preferred_element_type=_f32)
    t = jnp.dot(x[...], n2a[...], preferred_element_type=_f32)
    t += jnp.dot(agg, n2b[...], preferred_element_type=_f32)
    t += jnp.dot(ub, n2c[...], preferred_element_type=_f32) + b1n2[...]
    t = jnp.maximum(t, 0.0)
    xn = jnp.dot(t, n2w2[...], preferred_element_type=_f32) + b2n2[...]
    xo[...] = xn

    @pl.when(pl.program_id(0) == 0)
    def _():
        xmo[...] = jnp.zeros_like(xmo)

    xmo[...] += lax.dot_general(oh[...], xn, (((0,), (0,)), ((), ())),
                                preferred_element_type=_f32)


def _node_call(S, x, oh, cnt, u, w2a, w2b, b2n1, n2a, n2b, n2c, b1n2,
               n2w2, b2n2):
    grid = (NPAD // BN,)
    wspec = lambda r, c: pl.BlockSpec((r, c), lambda i: (0, 0))
    return pl.pallas_call(
        _node_body,
        grid=grid,
        in_specs=[
            pl.BlockSpec((NC, BN, F), lambda i: (0, i, 0)),
            pl.BlockSpec((BN, F), lambda i: (i, 0)),
            pl.BlockSpec((BN, G), lambda i: (i, 0)),
            pl.BlockSpec((BN, 1), lambda i: (i, 0)),
            wspec(G, FG),
            wspec(F, H), wspec(F, H), wspec(1, H),
            wspec(F, H), wspec(H, H), wspec(FG, H), wspec(1, H),
            wspec(H, F), wspec(1, F),
        ],
        out_specs=[
            pl.BlockSpec((BN, F), lambda i: (i, 0)),
            pl.BlockSpec((G, F), lambda i: (0, 0)),
        ],
        out_shape=[
            jax.ShapeDtypeStruct((NPAD, F), _f32),
            jax.ShapeDtypeStruct((G, F), _f32),
        ],
        compiler_params=pltpu.CompilerParams(
            dimension_semantics=("arbitrary",)),
    )(S, x, oh, cnt, u, w2a, w2b, b2n1, n2a, n2b, n2c, b1n2, n2w2, b2n2)


def _glob_body(u, xms, gc, ga, gb, b1g, gw2, b2g, wd, b1e, uo, po):
    xm = xms[...] / jnp.maximum(gc[...], 1.0)
    t = jnp.dot(u[...], ga[...], preferred_element_type=_f32)
    t += jnp.dot(xm, gb[...], preferred_element_type=_f32) + b1g[...]
    t = jnp.maximum(t, 0.0)
    un = jnp.dot(t, gw2[...], preferred_element_type=_f32) + b2g[...]
    uo[...] = un
    po[...] = jnp.dot(un, wd[...], preferred_element_type=_f32) + b1e[...]


def _glob_call(u, xms, gc, ga, gb, b1g, gw2, b2g, wd, b1e):
    wspec = lambda r, c: pl.BlockSpec((r, c), lambda i: (0, 0))
    return pl.pallas_call(
        _glob_body,
        grid=(1,),
        in_specs=[
            wspec(G, FG), wspec(G, F), wspec(G, 1),
            wspec(FG, H), wspec(F, H), wspec(1, H),
            wspec(H, FG), wspec(1, FG),
            wspec(FG, H), wspec(1, H),
        ],
        out_specs=[
            pl.BlockSpec((G, FG), lambda i: (0, 0)),
            pl.BlockSpec((G, H), lambda i: (0, 0)),
        ],
        out_shape=[
            jax.ShapeDtypeStruct((G, FG), _f32),
            jax.ShapeDtypeStruct((G, H), _f32),
        ],
        compiler_params=pltpu.CompilerParams(
            dimension_semantics=("arbitrary",)),
    )(u, xms, gc, ga, gb, b1g, gw2, b2g, wd, b1e)


def kernel(x, edge_index, edge_attr, u, batch,
           edge_w1, edge_b1, edge_w2, edge_b2,
           node1_w1, node1_b1, node1_w2, node1_b2,
           node2_w1, node2_b1, node2_w2, node2_b2,
           glob_w1, glob_b1, glob_w2, glob_b2):
    row = edge_index[0].astype(_i32)
    col = edge_index[1].astype(_i32)
    row2 = jnp.pad(row, (0, GPAD * CH - E),
                   constant_values=N).reshape(GPAD, CH)
    col2 = jnp.pad(col, (0, GPAD * CH - E),
                   constant_values=N).reshape(GPAD, CH)
    xt = jnp.pad(x, ((0, NPAD - N), (0, 0)))
    b2 = jnp.pad(batch.astype(_i32), (0, NPAD - N),
                 constant_values=G).reshape(NPAD, 1)
    ea = jnp.pad(edge_attr, ((0, EP - E), (0, 0)))

    wxx = edge_w1[:2 * F]
    wc = edge_w1[2 * F:2 * F + FE]
    wd = edge_w1[2 * F + FE:]
    b1e = edge_b1.reshape(1, H)
    b2e = edge_b2.reshape(1, FE)
    wna = node1_w1[:F]
    wnb = node1_w1[F:]
    b1n = node1_b1.reshape(1, H)
    w2a = node1_w2[:F]
    w2b = node1_w2[F:]
    b2n1 = node1_b2.reshape(1, H)
    n2a = node2_w1[:F]
    n2b = node2_w1[F:F + H]
    n2c = node2_w1[F + H:]
    b1n2 = node2_b1.reshape(1, H)
    b2n2 = node2_b2.reshape(1, F)
    ga = glob_w1[:FG]
    gb = glob_w1[FG:]
    b1g = glob_b1.reshape(1, H)
    b2g = glob_b2.reshape(1, FG)

    bat = jnp.pad(batch.astype(_i32), (0, NPAD - N), constant_values=0)
    b128 = jnp.broadcast_to(bat[:, None], (NPAD, F))
    cntp = _count_call(col2)
    gw = _gidx_call(row2, b128)
    gi = _compact_call(gw)
    oh, pmat, cnt, gc = _prep_call(b2, cntp, u, wd, b1e)

    for _ in range(3):
        xr, xc = _gather_call(xt, row2, col2)
        ea, h3 = _edge_call(xr, xc, gi, ea, wxx, wc, pmat, edge_w2, b2e,
                            wna, wnb, b1n)
        S = _scatter_call(h3, col2)
        xt, xms = _node_call(S, xt, oh, cnt, u, w2a, w2b, b2n1,
                             n2a, n2b, n2c, b1n2, node2_w2, b2n2)
        u, pmat = _glob_call(u, xms, gc, ga, gb, b1g, glob_w2, b2g, wd, b1e)

    return xt[:N], ea[:E], u


# speedup vs baseline: 1.3353x; 1.0585x over previous
import functools

import jax
import jax.numpy as jnp
from jax import lax
from jax.experimental import pallas as pl
from jax.experimental.pallas import tpu as pltpu
from jax.experimental.pallas import tpu_sc as plsc

N = 10000
E = 160000
F = 128
FE = 16
FG = 16
H = 256
G = 64

NC = 2
NS = 16
NW = NC * NS
CH = 128
EP = 163840
NCH = EP // (NW * CH)
SCH = EP // (NS * CH)
NPAD = 10240
NROW = NPAD // NS
BE = 2048
BN = 1024

_f32 = jnp.float32
_i32 = jnp.int32


def _mesh():
    return plsc.VectorSubcoreMesh(core_axis_name="c", subcore_axis_name="s",
                                  num_cores=NC, num_subcores=NS)


def _count_body(col2, cntp, coli, buf, obuf, acc):
    c = lax.axis_index("c")
    s = lax.axis_index("s")
    w = s * NC + c
    zero16 = jnp.zeros((16,), _f32)
    one16 = jnp.ones((16,), _f32)

    def zb(i, carry):
        for j in range(F // 16):
            buf[i, pl.ds(j * 16, 16)] = zero16
            obuf[i, pl.ds(j * 16, 16)] = one16
        return carry

    lax.fori_loop(0, CH, zb, 0)

    def zc(k, carry):
        pltpu.sync_copy(buf, acc.at[pl.ds(s * NROW + k * CH, CH)])
        return carry

    lax.fori_loop(0, NROW // CH, zc, 0)
    plsc.subcore_barrier()
    pltpu.sync_copy(col2.at[pl.ds(w * NCH, NCH)], coli)

    def step(i, carry):
        pltpu.sync_copy(obuf, acc.at[coli.at[i]], add=True)
        return carry

    lax.fori_loop(0, NCH, step, 0)
    plsc.subcore_barrier()
    pltpu.sync_copy(acc.at[pl.ds(s * NROW, NROW)],
                    cntp.at[c, pl.ds(s * NROW, NROW)])


def _count_call(col2):
    k = pl.kernel(
        _count_body,
        out_type=jax.ShapeDtypeStruct((NC, NPAD, F), _f32),
        mesh=_mesh(),
        scratch_types=[
            pltpu.VMEM((NCH, CH), _i32),
            pltpu.VMEM((CH, F), _f32),
            pltpu.VMEM((CH, F), _f32),
            pltpu.VMEM_SHARED((NPAD, F), _f32),
        ],
    )
    return k(col2)


def _gidx_body(row2, b128, gw, rowi, bufg0, bufg1, s0, s1):
    c = lax.axis_index("c")
    s = lax.axis_index("s")
    w = s * NC + c
    pltpu.sync_copy(row2.at[pl.ds(w * NCH, NCH)], rowi)

    def start(l, b, sv):
        pltpu.async_copy(b128.at[rowi.at[l]], b, sv)

    def finish(l, b, sv):
        pltpu.make_async_copy(b128.at[rowi.at[l]], b, sv).wait()
        pltpu.sync_copy(b, gw.at[pl.ds((w * NCH + l) * CH, CH)])

    start(0, bufg0, s0)
    start(1, bufg1, s1)

    def step(k, carry):
        i0 = 2 * k
        finish(i0, bufg0, s0)
        start(i0 + 2, bufg0, s0)
        finish(i0 + 1, bufg1, s1)
        start(i0 + 3, bufg1, s1)
        return carry

    lax.fori_loop(0, NCH // 2 - 1, step, 0)
    finish(NCH - 2, bufg0, s0)
    finish(NCH - 1, bufg1, s1)


def _gidx_call(row2, b128):
    k = pl.kernel(
        _gidx_body,
        out_type=jax.ShapeDtypeStruct((EP, F), _i32),
        mesh=_mesh(),
        scratch_types=[
            pltpu.VMEM((NCH, CH), _i32),
            pltpu.VMEM((CH, F), _i32),
            pltpu.VMEM((CH, F), _i32),
            pltpu.SemaphoreType.DMA,
            pltpu.SemaphoreType.DMA,
        ],
    )
    return k(row2, b128)


def _compact_body(gw, gi_o):
    gi_o[...] = gw[...][:, :1]


def _compact_call(gw):
    return pl.pallas_call(
        _compact_body,
        grid=(EP // BE,),
        in_specs=[pl.BlockSpec((BE, F), lambda i: (i, 0))],
        out_specs=pl.BlockSpec((BE, 1), lambda i: (i, 0)),
        out_shape=jax.ShapeDtypeStruct((EP, 1), _i32),
        compiler_params=pltpu.CompilerParams(
            dimension_semantics=("arbitrary",)),
    )(gw)


GA = 72
GB = 8
GPAD = 1344


def _gather_body(xtab, row2, col2, xr_o, xc_o,
                 rowi, coli, bxr0, bxc0, bxr1, bxc1, sr0, sc0, sr1, sc1):
    c = lax.axis_index("c")
    s = lax.axis_index("s")
    nch = jnp.where(c == 0, GA, GB)
    cbase = jnp.where(c == 0, s * GA, 16 * GA + s * GB)
    pltpu.sync_copy(row2.at[pl.ds(cbase, GA)], rowi)
    pltpu.sync_copy(col2.at[pl.ds(cbase, GA)], coli)

    def start(l, br, bc, svr, svc):
        pltpu.async_copy(xtab.at[rowi.at[l]], br, svr)
        pltpu.async_copy(xtab.at[coli.at[l]], bc, svc)

    def finish(l, br, bc, svr, svc):
        pltpu.make_async_copy(xtab.at[rowi.at[l]], br, svr).wait()
        pltpu.make_async_copy(xtab.at[coli.at[l]], bc, svc).wait()
        base = (cbase + l) * CH
        pltpu.sync_copy(br, xr_o.at[pl.ds(base, CH)])
        pltpu.sync_copy(bc, xc_o.at[pl.ds(base, CH)])

    start(0, bxr0, bxc0, sr0, sc0)
    start(1, bxr1, bxc1, sr1, sc1)

    def step(k, carry):
        i0 = 2 * k
        finish(i0, bxr0, bxc0, sr0, sc0)
        start(i0 + 2, bxr0, bxc0, sr0, sc0)
        finish(i0 + 1, bxr1, bxc1, sr1, sc1)
        start(i0 + 3, bxr1, bxc1, sr1, sc1)
        return carry

    lax.fori_loop(0, nch // 2 - 1, step, 0)
    finish(nch - 2, bxr0, bxc0, sr0, sc0)
    finish(nch - 1, bxr1, bxc1, sr1, sc1)


def _gather_call(xtab, row2p, col2p):
    k = pl.kernel(
        _gather_body,
        out_type=[
            jax.ShapeDtypeStruct((EP, F), _f32),
            jax.ShapeDtypeStruct((EP, F), _f32),
        ],
        mesh=_mesh(),
        scratch_types=[
            pltpu.VMEM((GA, CH), _i32),
            pltpu.VMEM((GA, CH), _i32),
            pltpu.VMEM((CH, F), _f32),
            pltpu.VMEM((CH, F), _f32),
            pltpu.VMEM((CH, F), _f32),
            pltpu.VMEM((CH, F), _f32),
            pltpu.SemaphoreType.DMA,
            pltpu.SemaphoreType.DMA,
            pltpu.SemaphoreType.DMA,
            pltpu.SemaphoreType.DMA,
        ],
    )
    return k(xtab, row2p, col2p)


def _scatter_body(h3, col2, s_out, coli, buf0, buf1, acc, s0, s1):
    c = lax.axis_index("c")
    s = lax.axis_index("s")
    zero16 = jnp.zeros((16,), _f32)

    def zb(i, carry):
        for j in range(F // 16):
            buf0[i, pl.ds(j * 16, 16)] = zero16
        return carry

    lax.fori_loop(0, CH, zb, 0)

    def zc(k, carry):
        pltpu.sync_copy(buf0, acc.at[pl.ds(s * NROW + k * CH, CH)])
        return carry

    lax.fori_loop(0, NROW // CH, zc, 0)
    plsc.subcore_barrier()

    pltpu.sync_copy(col2.at[pl.ds(s * SCH, SCH)], coli)

    def start(l, b, sv):
        pltpu.async_copy(h3.at[c, pl.ds((s * SCH + l) * CH, CH)], b, sv)

    def finish(l, b, sv):
        pltpu.make_async_copy(h3.at[c, pl.ds((s * SCH + l) * CH, CH)],
                              b, sv).wait()
        pltpu.sync_copy(b, acc.at[coli.at[l]], add=True)

    start(0, buf0, s0)
    start(1, buf1, s1)

    def step(k, carry):
        i0 = 2 * k
        finish(i0, buf0, s0)
        start(i0 + 2, buf0, s0)
        finish(i0 + 1, buf1, s1)
        start(i0 + 3, buf1, s1)
        return carry

    lax.fori_loop(0, SCH // 2 - 1, step, 0)
    finish(SCH - 2, buf0, s0)
    finish(SCH - 1, buf1, s1)
    plsc.subcore_barrier()
    pltpu.sync_copy(acc.at[pl.ds(s * NROW, NROW)],
                    s_out.at[c, pl.ds(s * NROW, NROW)])


def _scatter_call(h3, col2):
    k = pl.kernel(
        _scatter_body,
        out_type=jax.ShapeDtypeStruct((NC, NPAD, F), _f32),
        mesh=_mesh(),
        scratch_types=[
            pltpu.VMEM((SCH, CH), _i32),
            pltpu.VMEM((CH, F), _f32),
            pltpu.VMEM((CH, F), _f32),
            pltpu.VMEM_SHARED((NPAD, F), _f32),
            pltpu.SemaphoreType.DMA,
            pltpu.SemaphoreType.DMA,
        ],
    )
    return k(h3, col2)


def _prep_body(b2, cntp, u, wd, b1e, oh_o, p_o, cnt_o, gc_o):
    oh = (b2[...] == lax.broadcasted_iota(_i32, (1, G), 1)).astype(_f32)
    oh_o[...] = oh
    p_o[...] = jnp.dot(u[...], wd[...], preferred_element_type=_f32) + b1e[...]
    cnt_o[...] = cntp[0][:, :1] + cntp[1][:, :1]

    @pl.when(pl.program_id(0) == 0)
    def _():
        gc_o[...] = jnp.zeros_like(gc_o)

    gc_o[...] += lax.dot_general(oh, jnp.ones((BN, 1), _f32),
                                 (((0,), (0,)), ((), ())),
                                 preferred_element_type=_f32)


def _prep_call(b2, cntp, u, wd, b1e):
    grid = (NPAD // BN,)
    return pl.pallas_call(
        _prep_body,
        grid=grid,
        in_specs=[
            pl.BlockSpec((BN, 1), lambda i: (i, 0)),
            pl.BlockSpec((NC, BN, F), lambda i: (0, i, 0)),
            pl.BlockSpec((G, FG), lambda i: (0, 0)),
            pl.BlockSpec((FG, H), lambda i: (0, 0)),
            pl.BlockSpec((1, H), lambda i: (0, 0)),
        ],
        out_specs=[
            pl.BlockSpec((BN, G), lambda i: (i, 0)),
            pl.BlockSpec((G, H), lambda i: (0, 0)),
            pl.BlockSpec((BN, 1), lambda i: (i, 0)),
            pl.BlockSpec((G, 1), lambda i: (0, 0)),
        ],
        out_shape=[
            jax.ShapeDtypeStruct((NPAD, G), _f32),
            jax.ShapeDtypeStruct((G, H), _f32),
            jax.ShapeDtypeStruct((NPAD, 1), _f32),
            jax.ShapeDtypeStruct((G, 1), _f32),
        ],
        compiler_params=pltpu.CompilerParams(
            dimension_semantics=("arbitrary",)),
    )(b2, cntp, u, wd, b1e)


def _edge_body(xr, xc, gi, ea, wxx, wc, pmat, w2e, b2e, wna, wnb, b1n,
               ea_o, h3_o):
    xx = jnp.concatenate([xr[...], xc[...]], axis=1)
    ohe = (gi[...] == lax.broadcasted_iota(_i32, (1, G), 1)).astype(_f32)
    h = jnp.dot(xx, wxx[...], preferred_element_type=_f32)
    h += jnp.dot(ea[...], wc[...], preferred_element_type=_f32)
    h += jnp.dot(ohe, pmat[...], preferred_element_type=_f32)
    h = jnp.maximum(h, 0.0)
    ean = jnp.dot(h, w2e[...], preferred_element_type=_f32) + b2e[...]
    hn = jnp.dot(xr[...], wna[...], preferred_element_type=_f32)
    hn += jnp.dot(ean, wnb[...], preferred_element_type=_f32) + b1n[...]
    hn = jnp.maximum(hn, 0.0)
    ea_o[...] = ean
    h3_o[0] = hn[:, :F]
    h3_o[1] = hn[:, F:]


def _edge_call(xr, xc, gi, ea, wxx, wc, pmat, w2e, b2e, wna, wnb, b1n):
    grid = (EP // BE,)
    wspec = lambda r, c: pl.BlockSpec((r, c), lambda i: (0, 0))
    return pl.pallas_call(
        _edge_body,
        grid=grid,
        in_specs=[
            pl.BlockSpec((BE, F), lambda i: (i, 0)),
            pl.BlockSpec((BE, F), lambda i: (i, 0)),
            pl.BlockSpec((BE, 1), lambda i: (i, 0)),
            pl.BlockSpec((BE, FE), lambda i: (i, 0)),
            wspec(2 * F, H), wspec(FE, H), wspec(G, H),
            wspec(H, FE), wspec(1, FE),
            wspec(F, H), wspec(FE, H), wspec(1, H),
        ],
        out_specs=[
            pl.BlockSpec((BE, FE), lambda i: (i, 0)),
            pl.BlockSpec((NC, BE, F), lambda i: (0, i, 0)),
        ],
        out_shape=[
            jax.ShapeDtypeStruct((EP, FE), _f32),
            jax.ShapeDtypeStruct((NC, EP, F), _f32),
        ],
        compiler_params=pltpu.CompilerParams(
            dimension_semantics=("arbitrary",)),
    )(xr, xc, gi, ea, wxx, wc, pmat, w2e, b2e, wna, wnb, b1n)


def _node_body(S, x, oh, cnt, u, w2a, w2b, b2n1, n2a, n2b, n2c, b1n2,
               n2w2, b2n2, xo, xmo):
    cnt_ = cnt[...]
    agg = jnp.dot(S[0], w2a[...], preferred_element_type=_f32)
    agg += jnp.dot(S[1], w2b[...], preferred_element_type=_f32)
    agg = (agg + cnt_ * b2n1[...]) / jnp.maximum(cnt_, 1.0)
    ub = jnp.dot(oh[...], u[...], preferred_element_type=_f32)
    t = jnp.dot(x[...], n2a[...], preferred_element_type=_f32)
    t += jnp.dot(agg, n2b[...], preferred_element_type=_f32)
    t += jnp.dot(ub, n2c[...], preferred_element_type=_f32) + b1n2[...]
    t = jnp.maximum(t, 0.0)
    xn = jnp.dot(t, n2w2[...], preferred_element_type=_f32) + b2n2[...]
    xo[...] = xn

    @pl.when(pl.program_id(0) == 0)
    def _():
        xmo[...] = jnp.zeros_like(xmo)

    xmo[...] += lax.dot_general(oh[...], xn, (((0,), (0,)), ((), ())),
                                preferred_element_type=_f32)


def _node_call(S, x, oh, cnt, u, w2a, w2b, b2n1, n2a, n2b, n2c, b1n2,
               n2w2, b2n2):
    grid = (NPAD // BN,)
    wspec = lambda r, c: pl.BlockSpec((r, c), lambda i: (0, 0))
    return pl.pallas_call(
        _node_body,
        grid=grid,
        in_specs=[
            pl.BlockSpec((NC, BN, F), lambda i: (0, i, 0)),
            pl.BlockSpec((BN, F), lambda i: (i, 0)),
            pl.BlockSpec((BN, G), lambda i: (i, 0)),
            pl.BlockSpec((BN, 1), lambda i: (i, 0)),
            wspec(G, FG),
            wspec(F, H), wspec(F, H), wspec(1, H),
            wspec(F, H), wspec(H, H), wspec(FG, H), wspec(1, H),
            wspec(H, F), wspec(1, F),
        ],
        out_specs=[
            pl.BlockSpec((BN, F), lambda i: (i, 0)),
            pl.BlockSpec((G, F), lambda i: (0, 0)),
        ],
        out_shape=[
            jax.ShapeDtypeStruct((NPAD, F), _f32),
            jax.ShapeDtypeStruct((G, F), _f32),
        ],
        compiler_params=pltpu.CompilerParams(
            dimension_semantics=("arbitrary",)),
    )(S, x, oh, cnt, u, w2a, w2b, b2n1, n2a, n2b, n2c, b1n2, n2w2, b2n2)


def _glob_body(u, xms, gc, ga, gb, b1g, gw2, b2g, wd, b1e, uo, po):
    xm = xms[...] / jnp.maximum(gc[...], 1.0)
    t = jnp.dot(u[...], ga[...], preferred_element_type=_f32)
    t += jnp.dot(xm, gb[...], preferred_element_type=_f32) + b1g[...]
    t = jnp.maximum(t, 0.0)
    un = jnp.dot(t, gw2[...], preferred_element_type=_f32) + b2g[...]
    uo[...] = un
    po[...] = jnp.dot(un, wd[...], preferred_element_type=_f32) + b1e[...]


def _glob_call(u, xms, gc, ga, gb, b1g, gw2, b2g, wd, b1e):
    wspec = lambda r, c: pl.BlockSpec((r, c), lambda i: (0, 0))
    return pl.pallas_call(
        _glob_body,
        grid=(1,),
        in_specs=[
            wspec(G, FG), wspec(G, F), wspec(G, 1),
            wspec(FG, H), wspec(F, H), wspec(1, H),
            wspec(H, FG), wspec(1, FG),
            wspec(FG, H), wspec(1, H),
        ],
        out_specs=[
            pl.BlockSpec((G, FG), lambda i: (0, 0)),
            pl.BlockSpec((G, H), lambda i: (0, 0)),
        ],
        out_shape=[
            jax.ShapeDtypeStruct((G, FG), _f32),
            jax.ShapeDtypeStruct((G, H), _f32),
        ],
        compiler_params=pltpu.CompilerParams(
            dimension_semantics=("arbitrary",)),
    )(u, xms, gc, ga, gb, b1g, gw2, b2g, wd, b1e)


def kernel(x, edge_index, edge_attr, u, batch,
           edge_w1, edge_b1, edge_w2, edge_b2,
           node1_w1, node1_b1, node1_w2, node1_b2,
           node2_w1, node2_b1, node2_w2, node2_b2,
           glob_w1, glob_b1, glob_w2, glob_b2):
    row = edge_index[0].astype(_i32)
    col = edge_index[1].astype(_i32)
    row2 = jnp.pad(row, (0, GPAD * CH - E),
                   constant_values=N).reshape(GPAD, CH)
    col2 = jnp.pad(col, (0, GPAD * CH - E),
                   constant_values=N).reshape(GPAD, CH)
    xt = jnp.pad(x, ((0, NPAD - N), (0, 0)))
    b2 = jnp.pad(batch.astype(_i32), (0, NPAD - N),
                 constant_values=G).reshape(NPAD, 1)
    ea = jnp.pad(edge_attr, ((0, EP - E), (0, 0)))

    wxx = edge_w1[:2 * F]
    wc = edge_w1[2 * F:2 * F + FE]
    wd = edge_w1[2 * F + FE:]
    b1e = edge_b1.reshape(1, H)
    b2e = edge_b2.reshape(1, FE)
    wna = node1_w1[:F]
    wnb = node1_w1[F:]
    b1n = node1_b1.reshape(1, H)
    w2a = node1_w2[:F]
    w2b = node1_w2[F:]
    b2n1 = node1_b2.reshape(1, H)
    n2a = node2_w1[:F]
    n2b = node2_w1[F:F + H]
    n2c = node2_w1[F + H:]
    b1n2 = node2_b1.reshape(1, H)
    b2n2 = node2_b2.reshape(1, F)
    ga = glob_w1[:FG]
    gb = glob_w1[FG:]
    b1g = glob_b1.reshape(1, H)
    b2g = glob_b2.reshape(1, FG)

    bat = jnp.pad(batch.astype(_i32), (0, NPAD - N), constant_values=0)
    b128 = jnp.broadcast_to(bat[:, None], (NPAD, F))
    cntp = _count_call(col2)
    gw = _gidx_call(row2, b128)
    gi = _compact_call(gw)
    oh, pmat, cnt, gc = _prep_call(b2, cntp, u, wd, b1e)

    for _ in range(3):
        xr, xc = _gather_call(xt, row2, col2)
        ea, h3 = _edge_call(xr, xc, gi, ea, wxx, wc, pmat, edge_w2, b2e,
                            wna, wnb, b1n)
        S = _scatter_call(h3, col2)
        xt, xms = _node_call(S, xt, oh, cnt, u, w2a, w2b, b2n1,
                             n2a, n2b, n2c, b1n2, node2_w2, b2n2)
        u, pmat = _glob_call(u, xms, gc, ga, gb, b1g, glob_w2, b2g, wd, b1e)

    return xt[:N], ea[:E], u


# speedup vs baseline: 1.3365x; 1.0009x over previous
import jax
import jax.numpy as jnp
from jax import lax
from jax.experimental import pallas as pl
from jax.experimental.pallas import tpu as pltpu
from jax.experimental.pallas import tpu_sc as plsc

N = 10000
E = 160000
F = 128
FE = 16
FG = 16
H = 256
G = 64

NC = 2
NS = 16
NW = NC * NS
CH = 128
EP = 163840
NCH = EP // (NW * CH)
SCH = EP // (NS * CH)
NPAD = 10240
NROW = NPAD // NS
BE = 2048
BN = 1024

_f32 = jnp.float32
_i32 = jnp.int32


def _mesh():
    return plsc.VectorSubcoreMesh(core_axis_name="c", subcore_axis_name="s",
                                  num_cores=NC, num_subcores=NS)


def _count_body(col2, cntp, coli, buf, obuf, acc):
    c = lax.axis_index("c")
    s = lax.axis_index("s")
    w = s * NC + c
    zero16 = jnp.zeros((16,), _f32)
    one16 = jnp.ones((16,), _f32)

    def zb(i, carry):
        for j in range(F // 16):
            buf[i, pl.ds(j * 16, 16)] = zero16
            obuf[i, pl.ds(j * 16, 16)] = one16
        return carry

    lax.fori_loop(0, CH, zb, 0)

    def zc(k, carry):
        pltpu.sync_copy(buf, acc.at[pl.ds(s * NROW + k * CH, CH)])
        return carry

    lax.fori_loop(0, NROW // CH, zc, 0)
    plsc.subcore_barrier()
    pltpu.sync_copy(col2.at[pl.ds(w * NCH, NCH)], coli)

    def step(i, carry):
        pltpu.sync_copy(obuf, acc.at[coli.at[i]], add=True)
        return carry

    lax.fori_loop(0, NCH, step, 0)
    plsc.subcore_barrier()
    pltpu.sync_copy(acc.at[pl.ds(s * NROW, NROW)],
                    cntp.at[c, pl.ds(s * NROW, NROW)])


def _count_call(col2):
    k = pl.kernel(
        _count_body,
        out_type=jax.ShapeDtypeStruct((NC, NPAD, F), _f32),
        mesh=_mesh(),
        scratch_types=[
            pltpu.VMEM((NCH, CH), _i32),
            pltpu.VMEM((CH, F), _f32),
            pltpu.VMEM((CH, F), _f32),
            pltpu.VMEM_SHARED((NPAD, F), _f32),
        ],
    )
    return k(col2)


def _gidx_body(row2, b128, gw, rowi, bufg0, bufg1, s0, s1):
    c = lax.axis_index("c")
    s = lax.axis_index("s")
    w = s * NC + c
    pltpu.sync_copy(row2.at[pl.ds(w * NCH, NCH)], rowi)

    def start(l, b, sv):
        pltpu.async_copy(b128.at[rowi.at[l]], b, sv)

    def finish(l, b, sv):
        pltpu.make_async_copy(b128.at[rowi.at[l]], b, sv).wait()
        pltpu.sync_copy(b, gw.at[pl.ds((w * NCH + l) * CH, CH)])

    start(0, bufg0, s0)
    start(1, bufg1, s1)

    def step(k, carry):
        i0 = 2 * k
        finish(i0, bufg0, s0)
        start(i0 + 2, bufg0, s0)
        finish(i0 + 1, bufg1, s1)
        start(i0 + 3, bufg1, s1)
        return carry

    lax.fori_loop(0, NCH // 2 - 1, step, 0)
    finish(NCH - 2, bufg0, s0)
    finish(NCH - 1, bufg1, s1)


def _gidx_call(row2, b128):
    k = pl.kernel(
        _gidx_body,
        out_type=jax.ShapeDtypeStruct((EP, F), _i32),
        mesh=_mesh(),
        scratch_types=[
            pltpu.VMEM((NCH, CH), _i32),
            pltpu.VMEM((CH, F), _i32),
            pltpu.VMEM((CH, F), _i32),
            pltpu.SemaphoreType.DMA,
            pltpu.SemaphoreType.DMA,
        ],
    )
    return k(row2, b128)


def _compact_body(gw, gi_o):
    gi_o[...] = gw[...][:, :1]


def _compact_call(gw):
    return pl.pallas_call(
        _compact_body,
        grid=(EP // BE,),
        in_specs=[pl.BlockSpec((BE, F), lambda i: (i, 0))],
        out_specs=pl.BlockSpec((BE, 1), lambda i: (i, 0)),
        out_shape=jax.ShapeDtypeStruct((EP, 1), _i32),
        compiler_params=pltpu.CompilerParams(
            dimension_semantics=("arbitrary",)),
    )(gw)


GA = 72
GB = 8
GPAD = 1344


def _gather_body(xtab, row2, col2, xr_o, xc_o,
                 rowi, coli, bxr0, bxc0, bxr1, bxc1, sr0, sc0, sr1, sc1):
    c = lax.axis_index("c")
    s = lax.axis_index("s")
    nch = jnp.where(c == 0, GA, GB)
    cbase = jnp.where(c == 0, s * GA, 16 * GA + s * GB)
    pltpu.sync_copy(row2.at[pl.ds(cbase, GA)], rowi)
    pltpu.sync_copy(col2.at[pl.ds(cbase, GA)], coli)

    def start(l, br, bc, svr, svc):
        pltpu.async_copy(xtab.at[rowi.at[l]], br, svr)
        pltpu.async_copy(xtab.at[coli.at[l]], bc, svc)

    def finish(l, br, bc, svr, svc):
        pltpu.make_async_copy(xtab.at[rowi.at[l]], br, svr).wait()
        pltpu.make_async_copy(xtab.at[coli.at[l]], bc, svc).wait()
        base = (cbase + l) * CH
        pltpu.sync_copy(br, xr_o.at[pl.ds(base, CH)])
        pltpu.sync_copy(bc, xc_o.at[pl.ds(base, CH)])

    start(0, bxr0, bxc0, sr0, sc0)
    start(1, bxr1, bxc1, sr1, sc1)

    def step(k, carry):
        i0 = 2 * k
        finish(i0, bxr0, bxc0, sr0, sc0)
        start(i0 + 2, bxr0, bxc0, sr0, sc0)
        finish(i0 + 1, bxr1, bxc1, sr1, sc1)
        start(i0 + 3, bxr1, bxc1, sr1, sc1)
        return carry

    lax.fori_loop(0, nch // 2 - 1, step, 0)
    finish(nch - 2, bxr0, bxc0, sr0, sc0)
    finish(nch - 1, bxr1, bxc1, sr1, sc1)


def _gather_call(xtab, row2p, col2p):
    k = pl.kernel(
        _gather_body,
        out_type=[
            jax.ShapeDtypeStruct((EP, F), _f32),
            jax.ShapeDtypeStruct((EP, F), _f32),
        ],
        mesh=_mesh(),
        scratch_types=[
            pltpu.VMEM((GA, CH), _i32),
            pltpu.VMEM((GA, CH), _i32),
            pltpu.VMEM((CH, F), _f32),
            pltpu.VMEM((CH, F), _f32),
            pltpu.VMEM((CH, F), _f32),
            pltpu.VMEM((CH, F), _f32),
            pltpu.SemaphoreType.DMA,
            pltpu.SemaphoreType.DMA,
            pltpu.SemaphoreType.DMA,
            pltpu.SemaphoreType.DMA,
        ],
    )
    return k(xtab, row2p, col2p)


def _scatter_body(h3, col2, s_out, coli, buf0, buf1, acc, s0, s1):
    c = lax.axis_index("c")
    s = lax.axis_index("s")
    zero16 = jnp.zeros((16,), _f32)

    def zb(i, carry):
        for j in range(F // 16):
            buf0[i, pl.ds(j * 16, 16)] = zero16
        return carry

    lax.fori_loop(0, CH, zb, 0)

    def zc(k, carry):
        pltpu.sync_copy(buf0, acc.at[pl.ds(s * NROW + k * CH, CH)])
        return carry

    lax.fori_loop(0, NROW // CH, zc, 0)
    plsc.subcore_barrier()

    pltpu.sync_copy(col2.at[pl.ds(s * SCH, SCH)], coli)

    def start(l, b, sv):
        pltpu.async_copy(h3.at[c, pl.ds((s * SCH + l) * CH, CH)], b, sv)

    def finish(l, b, sv):
        pltpu.make_async_copy(h3.at[c, pl.ds((s * SCH + l) * CH, CH)],
                              b, sv).wait()
        pltpu.sync_copy(b, acc.at[coli.at[l]], add=True)

    start(0, buf0, s0)
    start(1, buf1, s1)

    def step(k, carry):
        i0 = 2 * k
        finish(i0, buf0, s0)
        start(i0 + 2, buf0, s0)
        finish(i0 + 1, buf1, s1)
        start(i0 + 3, buf1, s1)
        return carry

    lax.fori_loop(0, SCH // 2 - 1, step, 0)
    finish(SCH - 2, buf0, s0)
    finish(SCH - 1, buf1, s1)
    plsc.subcore_barrier()
    pltpu.sync_copy(acc.at[pl.ds(s * NROW, NROW)],
                    s_out.at[c, pl.ds(s * NROW, NROW)])


def _scatter_call(h3, col2):
    k = pl.kernel(
        _scatter_body,
        out_type=jax.ShapeDtypeStruct((NC, NPAD, F), _f32),
        mesh=_mesh(),
        scratch_types=[
            pltpu.VMEM((SCH, CH), _i32),
            pltpu.VMEM((CH, F), _f32),
            pltpu.VMEM((CH, F), _f32),
            pltpu.VMEM_SHARED((NPAD, F), _f32),
            pltpu.SemaphoreType.DMA,
            pltpu.SemaphoreType.DMA,
        ],
    )
    return k(h3, col2)


def _prep_body(b2, cntp, u, wd, b1e, oh_o, p_o, cnt_o, gc_o):
    oh = (b2[...] == lax.broadcasted_iota(_i32, (1, G), 1)).astype(_f32)
    oh_o[...] = oh
    p_o[...] = jnp.dot(u[...], wd[...], preferred_element_type=_f32) + b1e[...]
    cnt_o[...] = cntp[0][:, :1] + cntp[1][:, :1]

    @pl.when(pl.program_id(0) == 0)
    def _():
        gc_o[...] = jnp.zeros_like(gc_o)

    gc_o[...] += lax.dot_general(oh, jnp.ones((BN, 1), _f32),
                                 (((0,), (0,)), ((), ())),
                                 preferred_element_type=_f32)


def _prep_call(b2, cntp, u, wd, b1e):
    grid = (NPAD // BN,)
    return pl.pallas_call(
        _prep_body,
        grid=grid,
        in_specs=[
            pl.BlockSpec((BN, 1), lambda i: (i, 0)),
            pl.BlockSpec((NC, BN, F), lambda i: (0, i, 0)),
            pl.BlockSpec((G, FG), lambda i: (0, 0)),
            pl.BlockSpec((FG, H), lambda i: (0, 0)),
            pl.BlockSpec((1, H), lambda i: (0, 0)),
        ],
        out_specs=[
            pl.BlockSpec((BN, G), lambda i: (i, 0)),
            pl.BlockSpec((G, H), lambda i: (0, 0)),
            pl.BlockSpec((BN, 1), lambda i: (i, 0)),
            pl.BlockSpec((G, 1), lambda i: (0, 0)),
        ],
        out_shape=[
            jax.ShapeDtypeStruct((NPAD, G), _f32),
            jax.ShapeDtypeStruct((G, H), _f32),
            jax.ShapeDtypeStruct((NPAD, 1), _f32),
            jax.ShapeDtypeStruct((G, 1), _f32),
        ],
        compiler_params=pltpu.CompilerParams(
            dimension_semantics=("arbitrary",)),
    )(b2, cntp, u, wd, b1e)


def _edge_body(xr, xc, gi, ea, wxx, wc, pmat, w2e, b2e, wna, wnb, b1n,
               ea_o, h3_o):
    xx = jnp.concatenate([xr[...], xc[...]], axis=1)
    ohe = (gi[...] == lax.broadcasted_iota(_i32, (1, G), 1)).astype(_f32)
    h = jnp.dot(xx, wxx[...], preferred_element_type=_f32)
    h += jnp.dot(ea[...], wc[...], preferred_element_type=_f32)
    h += jnp.dot(ohe, pmat[...], preferred_element_type=_f32)
    h = jnp.maximum(h, 0.0)
    ean = jnp.dot(h, w2e[...], preferred_element_type=_f32) + b2e[...]
    hn = jnp.dot(xr[...], wna[...], preferred_element_type=_f32)
    hn += jnp.dot(ean, wnb[...], preferred_element_type=_f32) + b1n[...]
    hn = jnp.maximum(hn, 0.0)
    ea_o[...] = ean
    h3_o[0] = hn[:, :F]
    h3_o[1] = hn[:, F:]


def _edge_call(xr, xc, gi, ea, wxx, wc, pmat, w2e, b2e, wna, wnb, b1n):
    grid = (EP // BE,)
    wspec = lambda r, c: pl.BlockSpec((r, c), lambda i: (0, 0))
    return pl.pallas_call(
        _edge_body,
        grid=grid,
        in_specs=[
            pl.BlockSpec((BE, F), lambda i: (i, 0)),
            pl.BlockSpec((BE, F), lambda i: (i, 0)),
            pl.BlockSpec((BE, 1), lambda i: (i, 0)),
            pl.BlockSpec((BE, FE), lambda i: (i, 0)),
            wspec(2 * F, H), wspec(FE, H), wspec(G, H),
            wspec(H, FE), wspec(1, FE),
            wspec(F, H), wspec(FE, H), wspec(1, H),
        ],
        out_specs=[
            pl.BlockSpec((BE, FE), lambda i: (i, 0)),
            pl.BlockSpec((NC, BE, F), lambda i: (0, i, 0)),
        ],
        out_shape=[
            jax.ShapeDtypeStruct((EP, FE), _f32),
            jax.ShapeDtypeStruct((NC, EP, F), _f32),
        ],
        compiler_params=pltpu.CompilerParams(
            dimension_semantics=("arbitrary",)),
    )(xr, xc, gi, ea, wxx, wc, pmat, w2e, b2e, wna, wnb, b1n)


def _node_body(S, x, oh, cnt, u, w2a, w2b, b2n1, n2a, n2b, n2c, b1n2,
               n2w2, b2n2, xo, xmo):
    cnt_ = cnt[...]
    agg = jnp.dot(S[0], w2a[...], preferred_element_type=_f32)
    agg += jnp.dot(S[1], w2b[...], preferred_element_type=_f32)
    agg = (agg + cnt_ * b2n1[...]) / jnp.maximum(cnt_, 1.0)
    ub = jnp.dot(oh[...], u[...], preferred_element_type=_f32)
    t = jnp.dot(x[...], n2a[...], preferred_element_type=_f32)
    t += jnp.dot(agg, n2b[...], preferred_element_type=_f32)
    t += jnp.dot(ub, n2c[...], preferred_element_type=_f32) + b1n2[...]
    t = jnp.maximum(t, 0.0)
    xn = jnp.dot(t, n2w2[...], preferred_element_type=_f32) + b2n2[...]
    xo[...] = xn

    @pl.when(pl.program_id(0) == 0)
    def _():
        xmo[...] = jnp.zeros_like(xmo)

    xmo[...] += lax.dot_general(oh[...], xn, (((0,), (0,)), ((), ())),
                                preferred_element_type=_f32)


def _node_call(S, x, oh, cnt, u, w2a, w2b, b2n1, n2a, n2b, n2c, b1n2,
               n2w2, b2n2):
    grid = (NPAD // BN,)
    wspec = lambda r, c: pl.BlockSpec((r, c), lambda i: (0, 0))
    return pl.pallas_call(
        _node_body,
        grid=grid,
        in_specs=[
            pl.BlockSpec((NC, BN, F), lambda i: (0, i, 0)),
            pl.BlockSpec((BN, F), lambda i: (i, 0)),
            pl.BlockSpec((BN, G), lambda i: (i, 0)),
            pl.BlockSpec((BN, 1), lambda i: (i, 0)),
            wspec(G, FG),
            wspec(F, H), wspec(F, H), wspec(1, H),
            wspec(F, H), wspec(H, H), wspec(FG, H), wspec(1, H),
            wspec(H, F), wspec(1, F),
        ],
        out_specs=[
            pl.BlockSpec((BN, F), lambda i: (i, 0)),
            pl.BlockSpec((G, F), lambda i: (0, 0)),
        ],
        out_shape=[
            jax.ShapeDtypeStruct((NPAD, F), _f32),
            jax.ShapeDtypeStruct((G, F), _f32),
        ],
        compiler_params=pltpu.CompilerParams(
            dimension_semantics=("arbitrary",)),
    )(S, x, oh, cnt, u, w2a, w2b, b2n1, n2a, n2b, n2c, b1n2, n2w2, b2n2)


def _glob_body(u, xms, gc, ga, gb, b1g, gw2, b2g, wd, b1e, uo, po):
    xm = xms[...] / jnp.maximum(gc[...], 1.0)
    t = jnp.dot(u[...], ga[...], preferred_element_type=_f32)
    t += jnp.dot(xm, gb[...], preferred_element_type=_f32) + b1g[...]
    t = jnp.maximum(t, 0.0)
    un = jnp.dot(t, gw2[...], preferred_element_type=_f32) + b2g[...]
    uo[...] = un
    po[...] = jnp.dot(un, wd[...], preferred_element_type=_f32) + b1e[...]


def _glob_call(u, xms, gc, ga, gb, b1g, gw2, b2g, wd, b1e):
    wspec = lambda r, c: pl.BlockSpec((r, c), lambda i: (0, 0))
    return pl.pallas_call(
        _glob_body,
        grid=(1,),
        in_specs=[
            wspec(G, FG), wspec(G, F), wspec(G, 1),
            wspec(FG, H), wspec(F, H), wspec(1, H),
            wspec(H, FG), wspec(1, FG),
            wspec(FG, H), wspec(1, H),
        ],
        out_specs=[
            pl.BlockSpec((G, FG), lambda i: (0, 0)),
            pl.BlockSpec((G, H), lambda i: (0, 0)),
        ],
        out_shape=[
            jax.ShapeDtypeStruct((G, FG), _f32),
            jax.ShapeDtypeStruct((G, H), _f32),
        ],
        compiler_params=pltpu.CompilerParams(
            dimension_semantics=("arbitrary",)),
    )(u, xms, gc, ga, gb, b1g, gw2, b2g, wd, b1e)


def kernel(x, edge_index, edge_attr, u, batch,
           edge_w1, edge_b1, edge_w2, edge_b2,
           node1_w1, node1_b1, node1_w2, node1_b2,
           node2_w1, node2_b1, node2_w2, node2_b2,
           glob_w1, glob_b1, glob_w2, glob_b2):
    row = edge_index[0].astype(_i32)
    col = edge_index[1].astype(_i32)
    row2 = jnp.pad(row, (0, GPAD * CH - E),
                   constant_values=N).reshape(GPAD, CH)
    col2 = jnp.pad(col, (0, GPAD * CH - E),
                   constant_values=N).reshape(GPAD, CH)
    xt = jnp.pad(x, ((0, NPAD - N), (0, 0)))
    b2 = jnp.pad(batch.astype(_i32), (0, NPAD - N),
                 constant_values=G).reshape(NPAD, 1)
    ea = jnp.pad(edge_attr, ((0, EP - E), (0, 0)))

    wxx = edge_w1[:2 * F]
    wc = edge_w1[2 * F:2 * F + FE]
    wd = edge_w1[2 * F + FE:]
    b1e = edge_b1.reshape(1, H)
    b2e = edge_b2.reshape(1, FE)
    wna = node1_w1[:F]
    wnb = node1_w1[F:]
    b1n = node1_b1.reshape(1, H)
    w2a = node1_w2[:F]
    w2b = node1_w2[F:]
    b2n1 = node1_b2.reshape(1, H)
    n2a = node2_w1[:F]
    n2b = node2_w1[F:F + H]
    n2c = node2_w1[F + H:]
    b1n2 = node2_b1.reshape(1, H)
    b2n2 = node2_b2.reshape(1, F)
    ga = glob_w1[:FG]
    gb = glob_w1[FG:]
    b1g = glob_b1.reshape(1, H)
    b2g = glob_b2.reshape(1, FG)

    bat = jnp.pad(batch.astype(_i32), (0, NPAD - N), constant_values=0)
    b128 = jnp.broadcast_to(bat[:, None], (NPAD, F))
    cntp = _count_call(col2)
    gw = _gidx_call(row2, b128)
    gi = _compact_call(gw)
    oh, pmat, cnt, gc = _prep_call(b2, cntp, u, wd, b1e)

    for _ in range(3):
        xr, xc = _gather_call(xt, row2, col2)
        ea, h3 = _edge_call(xr, xc, gi, ea, wxx, wc, pmat, edge_w2, b2e,
                            wna, wnb, b1n)
        S = _scatter_call(h3, col2)
        xt, xms = _node_call(S, xt, oh, cnt, u, w2a, w2b, b2n1,
                             n2a, n2b, n2c, b1n2, node2_w2, b2n2)
        u, pmat = _glob_call(u, xms, gc, ga, gb, b1g, glob_w2, b2g, wd, b1e)

    return xt[:N], ea[:E], u
